# Initial kernel scaffold; baseline (speedup 1.0000x reference)
import jax
import jax.numpy as jnp
from jax.experimental import pallas as pl


def kernel(x_drug, x_gene, src_dd, dst_dd, src_dg, dst_dg, src_gd, dst_gd, src_gg, dst_gg, W1_dd, W1_dg, W1_gd, W1_gg, b1_drug, b1_gene, W2_dd, W2_dg, W2_gd, W2_gg, b2_drug, b2_gene):
    raise NotImplementedError("write your pallas kernel here")


# speedup vs baseline: 2.6895x; 2.6895x over previous
import functools

import jax
import jax.numpy as jnp
from jax import lax
from jax.experimental import pallas as pl
from jax.experimental.pallas import tpu as pltpu
from jax.experimental.pallas import tpu_sc as plsc

N = 50000
E = 150000
D_IN = 128
D_HID = 128
D_EMB = 64
CB = 32

NSUB = 16
NCORE = 2
NPAD = 50048
STRIPE = NPAD // NSUB
LAST_FLUSH = N - (NSUB - 1) * STRIPE
DUMP = N
EB = 600
NBATCH = 16
SLAB = EB * NBATCH
EP = SLAB * NSUB

f32 = jnp.float32
i32 = jnp.int32

_L1_TMAP = ((0, 1, 2, 3), (0, 1, 2, 3), (4, 5, 6, 7), (4, 5, 6, 7))
_L2_TMAP = ((0, 1), (2, 3), (4, 5), (6, 7))


def _make_agg(nblk_per_core, with_counts, tmap):
    nblk = nblk_per_core * NCORE
    mesh = plsc.VectorSubcoreMesh(core_axis_name="c", subcore_axis_name="s",
                                  num_cores=NCORE, num_subcores=NSUB)
    out_type = [jax.ShapeDtypeStruct((N, CB), f32) for _ in range(4 * nblk)]
    if with_counts:
        out_type += [jax.ShapeDtypeStruct((N,), f32) for _ in range(4)]
    scratch = [
        pltpu.VMEM_SHARED((NPAD, CB), f32),
        pltpu.VMEM((EB, CB), f32),
        pltpu.VMEM((EB,), i32),
        pltpu.VMEM((EB,), i32),
        pltpu.SemaphoreType.DMA,
    ]
    if with_counts:
        scratch += [
            pltpu.VMEM_SHARED((NPAD,), f32),
            pltpu.VMEM((EB,), f32),
            pltpu.VMEM((STRIPE,), f32),
        ]

    def body(*refs):
        tabs = refs[0:8]
        edges = refs[8:16]
        zc = refs[16]
        k = 17
        if with_counts:
            oc, zcn = refs[17], refs[18]
            k = 19
        outs = refs[k:k + 4 * nblk]
        k += 4 * nblk
        if with_counts:
            cnt_outs = refs[k:k + 4]
            k += 4
        acc, gbuf, sidx, didx, sem = refs[k:k + 5]
        if with_counts:
            cnt_acc, obuf, z1buf = refs[k + 5:k + 8]

        cid = lax.axis_index("c")
        sid = lax.axis_index("s")
        if with_counts:
            pltpu.sync_copy(oc, obuf)
            pltpu.sync_copy(zcn, z1buf)
        base = sid * STRIPE
        ebase = sid * SLAB

        for r in range(4):
            src_r = edges[2 * r]
            dst_r = edges[2 * r + 1]
            for p in range(nblk_per_core):
                for half in range(NCORE):
                    blk = half * nblk_per_core + p
                    tab = tabs[tmap[r][blk]]
                    out = outs[r * nblk + blk]
                    do_cnt = with_counts and blk == 0

                    def do_pass(tab=tab, out=out, src_r=src_r, dst_r=dst_r,
                                do_cnt=do_cnt, r=r):
                        pltpu.sync_copy(zc, gbuf)
                        for z in range(STRIPE // EB):
                            pltpu.sync_copy(
                                gbuf, acc.at[pl.ds(base + z * EB, EB), :])
                        rem = STRIPE % EB
                        pltpu.sync_copy(
                            gbuf.at[pl.ds(0, rem), :],
                            acc.at[pl.ds(base + STRIPE - rem, rem), :])
                        if do_cnt:
                            pltpu.sync_copy(
                                z1buf, cnt_acc.at[pl.ds(base, STRIPE)])
                        plsc.subcore_barrier()

                        def batch(b, carry):
                            off = ebase + b * EB
                            pltpu.sync_copy(src_r.at[pl.ds(off, EB)], sidx)
                            pltpu.sync_copy(dst_r.at[pl.ds(off, EB)], didx)
                            pltpu.async_copy(tab.at[sidx], gbuf, sem).wait()
                            pltpu.sync_copy(gbuf, acc.at[didx], add=True)
                            if do_cnt:
                                pltpu.sync_copy(
                                    obuf, cnt_acc.at[didx], add=True)
                            return carry

                        lax.fori_loop(0, NBATCH, batch, 0)
                        plsc.subcore_barrier()

                        def flush(flen):
                            def go():
                                pltpu.sync_copy(
                                    acc.at[pl.ds(base, flen), :],
                                    out.at[pl.ds(base, flen), :])
                                if do_cnt:
                                    pltpu.sync_copy(
                                        cnt_acc.at[pl.ds(base, flen)],
                                        cnt_outs[r].at[pl.ds(base, flen)])
                            return go

                        pl.when(sid < NSUB - 1)(flush(STRIPE))
                        pl.when(sid == NSUB - 1)(flush(LAST_FLUSH))
                        plsc.subcore_barrier()

                    pl.when(cid == half)(do_pass)

    return pl.kernel(body, out_type=tuple(out_type), mesh=mesh,
                     scratch_types=scratch,
                     compiler_params=pltpu.CompilerParams(
                         use_tc_tiling_on_sc=False))


_AGG_L1 = _make_agg(2, True, _L1_TMAP)
_AGG_L2 = _make_agg(1, False, _L2_TMAP)

_ROWS = 1000
_GRID = N // _ROWS


def _tc1_body(a0, a1, a2, a3, g0, g1, g2, g3, ca, cb, W1a, W1b, b1,
              W2x, W2y, tx0, tx1, ty0, ty1):
    ia = 1.0 / jnp.maximum(ca[...], 1.0)
    ib = 1.0 / jnp.maximum(cb[...], 1.0)
    h = jnp.broadcast_to(b1[...], (_ROWS, D_HID))
    for k, a in enumerate((a0, a1, a2, a3)):
        h = h + jnp.dot(a[...] * ia, W1a[k * CB:(k + 1) * CB, :],
                        preferred_element_type=f32)
    for k, g in enumerate((g0, g1, g2, g3)):
        h = h + jnp.dot(g[...] * ib, W1b[k * CB:(k + 1) * CB, :],
                        preferred_element_type=f32)
    h = jnp.maximum(h, 0.0)
    tx0[...] = jnp.dot(h, W2x[:, 0:CB], preferred_element_type=f32)
    tx1[...] = jnp.dot(h, W2x[:, CB:2 * CB], preferred_element_type=f32)
    ty0[...] = jnp.dot(h, W2y[:, 0:CB], preferred_element_type=f32)
    ty1[...] = jnp.dot(h, W2y[:, CB:2 * CB], preferred_element_type=f32)


def _tc1(Aa, Ab, ca, cb, W1a, W1b, b1, W2x, W2y):
    blk = lambda i: (i, 0)
    full = lambda i: (0, 0)
    spec_a = pl.BlockSpec((_ROWS, CB), blk)
    spec_c = pl.BlockSpec((_ROWS, 1), blk)
    return pl.pallas_call(
        _tc1_body,
        grid=(_GRID,),
        in_specs=[spec_a] * 8 + [spec_c] * 2 + [
            pl.BlockSpec((D_IN, D_HID), full),
            pl.BlockSpec((D_IN, D_HID), full),
            pl.BlockSpec((1, D_HID), full),
            pl.BlockSpec((D_HID, D_EMB), full),
            pl.BlockSpec((D_HID, D_EMB), full),
        ],
        out_specs=[spec_a] * 4,
        out_shape=[jax.ShapeDtypeStruct((N, CB), f32) for _ in range(4)],
    )(*Aa, *Ab, ca, cb, W1a, W1b, b1, W2x, W2y)


def _tc2_body(x0, x1, g0, g1, ca, cb, b2, out):
    ia = 1.0 / jnp.maximum(ca[...], 1.0)
    ib = 1.0 / jnp.maximum(cb[...], 1.0)
    out[...] = jnp.concatenate(
        [x0[...] * ia + g0[...] * ib, x1[...] * ia + g1[...] * ib],
        axis=1) + b2[...]


def _tc2(Ax, Ag, ca, cb, b2):
    blk = lambda i: (i, 0)
    full = lambda i: (0, 0)
    spec_a = pl.BlockSpec((_ROWS, CB), blk)
    spec_c = pl.BlockSpec((_ROWS, 1), blk)
    return pl.pallas_call(
        _tc2_body,
        grid=(_GRID,),
        in_specs=[spec_a] * 4 + [spec_c] * 2 + [pl.BlockSpec((1, D_EMB), full)],
        out_specs=pl.BlockSpec((_ROWS, D_EMB), blk),
        out_shape=jax.ShapeDtypeStruct((N, D_EMB), f32),
    )(*Ax, *Ag, ca, cb, b2)


def kernel(x_drug, x_gene, src_dd, dst_dd, src_dg, dst_dg, src_gd, dst_gd,
           src_gg, dst_gg, W1_dd, W1_dg, W1_gd, W1_gg, b1_drug, b1_gene,
           W2_dd, W2_dg, W2_gd, W2_gg, b2_drug, b2_gene):
    pad = EP - E
    spad = jnp.zeros((pad,), i32)
    dpad = jnp.full((pad,), DUMP, i32)
    edges = []
    for s, d in ((src_dd, dst_dd), (src_dg, dst_dg),
                 (src_gd, dst_gd), (src_gg, dst_gg)):
        edges.append(jnp.concatenate([s.astype(i32), spad]))
        edges.append(jnp.concatenate([d.astype(i32), dpad]))

    xblocks = [x_drug[:, k * CB:(k + 1) * CB] for k in range(4)]
    xblocks += [x_gene[:, k * CB:(k + 1) * CB] for k in range(4)]

    zc = jnp.zeros((EB, CB), f32)
    oc = jnp.ones((EB,), f32)
    zcn = jnp.zeros((STRIPE,), f32)

    outs = _AGG_L1(*xblocks, *edges, zc, oc, zcn)
    acc1 = [outs[4 * r:4 * (r + 1)] for r in range(4)]
    cnts = [c.reshape(N, 1) for c in outs[16:20]]

    t2_dd0, t2_dd1, t2_dg0, t2_dg1 = _tc1(
        acc1[0], acc1[2], cnts[0], cnts[2], W1_dd, W1_gd,
        b1_drug.reshape(1, D_HID), W2_dd, W2_dg)
    t2_gd0, t2_gd1, t2_gg0, t2_gg1 = _tc1(
        acc1[1], acc1[3], cnts[1], cnts[3], W1_dg, W1_gg,
        b1_gene.reshape(1, D_HID), W2_gd, W2_gg)

    outs2 = _AGG_L2(t2_dd0, t2_dd1, t2_dg0, t2_dg1,
                    t2_gd0, t2_gd1, t2_gg0, t2_gg1, *edges, zc)
    acc2 = [outs2[2 * r:2 * (r + 1)] for r in range(4)]

    o_drug = _tc2(acc2[0], acc2[2], cnts[0], cnts[2],
                  b2_drug.reshape(1, D_EMB))
    o_gene = _tc2(acc2[1], acc2[3], cnts[1], cnts[3],
                  b2_gene.reshape(1, D_EMB))
    return (o_drug, o_gene)


# speedup vs baseline: 2.9348x; 1.0912x over previous
import functools

import jax
import jax.numpy as jnp
from jax import lax
from jax.experimental import pallas as pl
from jax.experimental.pallas import tpu as pltpu
from jax.experimental.pallas import tpu_sc as plsc

N = 50000
E = 150000
D_IN = 128
D_HID = 128
D_EMB = 64
CB = 32

NSUB = 16
NCORE = 2
NPAD = 50048
STRIPE = NPAD // NSUB
LAST_FLUSH = N - (NSUB - 1) * STRIPE
DUMP = N
EB = 320
NBATCH = 30
SLAB = EB * NBATCH
EP = SLAB * NSUB

f32 = jnp.float32
i32 = jnp.int32

_L1_TMAP = ((0, 1, 2, 3), (0, 1, 2, 3), (4, 5, 6, 7), (4, 5, 6, 7))
_L2_TMAP = ((0, 1), (2, 3), (4, 5), (6, 7))


def _make_agg(nblk_per_core, with_counts, tmap):
    nblk = nblk_per_core * NCORE
    mesh = plsc.VectorSubcoreMesh(core_axis_name="c", subcore_axis_name="s",
                                  num_cores=NCORE, num_subcores=NSUB)
    out_type = [jax.ShapeDtypeStruct((N, CB), f32) for _ in range(4 * nblk)]
    if with_counts:
        out_type += [jax.ShapeDtypeStruct((N,), f32) for _ in range(4)]
    scratch = [
        pltpu.VMEM_SHARED((NPAD, CB), f32),
        pltpu.VMEM((EB, CB), f32),
        pltpu.VMEM((EB, CB), f32),
        pltpu.VMEM((EB,), i32),
        pltpu.VMEM((EB,), i32),
        pltpu.VMEM((EB,), i32),
        pltpu.VMEM((EB,), i32),
        pltpu.SemaphoreType.DMA,
        pltpu.SemaphoreType.DMA,
    ]
    if with_counts:
        scratch += [
            pltpu.VMEM_SHARED((NPAD,), f32),
            pltpu.VMEM((EB,), f32),
            pltpu.VMEM((STRIPE,), f32),
        ]

    def body(*refs):
        tabs = refs[0:8]
        edges = refs[8:16]
        zc = refs[16]
        k = 17
        if with_counts:
            oc, zcn = refs[17], refs[18]
            k = 19
        outs = refs[k:k + 4 * nblk]
        k += 4 * nblk
        if with_counts:
            cnt_outs = refs[k:k + 4]
            k += 4
        acc, gbuf0, gbuf1, sidx0, sidx1, didx0, didx1, sem0, sem1 = \
            refs[k:k + 9]
        if with_counts:
            cnt_acc, obuf, z1buf = refs[k + 9:k + 12]

        cid = lax.axis_index("c")
        sid = lax.axis_index("s")
        if with_counts:
            pltpu.sync_copy(oc, obuf)
            pltpu.sync_copy(zcn, z1buf)
        base = sid * STRIPE
        ebase = sid * SLAB

        for r in range(4):
            src_r = edges[2 * r]
            dst_r = edges[2 * r + 1]
            for p in range(nblk_per_core):
                for half in range(NCORE):
                    blk = half * nblk_per_core + p
                    tab = tabs[tmap[r][blk]]
                    out = outs[r * nblk + blk]
                    do_cnt = with_counts and blk == 0

                    def do_pass(tab=tab, out=out, src_r=src_r, dst_r=dst_r,
                                do_cnt=do_cnt, r=r):
                        pltpu.sync_copy(zc, gbuf0)
                        for z in range(STRIPE // EB):
                            pltpu.sync_copy(
                                gbuf0, acc.at[pl.ds(base + z * EB, EB), :])
                        rem = STRIPE % EB
                        pltpu.sync_copy(
                            gbuf0.at[pl.ds(0, rem), :],
                            acc.at[pl.ds(base + STRIPE - rem, rem), :])
                        if do_cnt:
                            pltpu.sync_copy(
                                z1buf, cnt_acc.at[pl.ds(base, STRIPE)])
                        plsc.subcore_barrier()

                        pltpu.sync_copy(src_r.at[pl.ds(ebase, EB)], sidx0)
                        pltpu.sync_copy(dst_r.at[pl.ds(ebase, EB)], didx0)
                        pltpu.async_copy(tab.at[sidx0], gbuf0, sem0)

                        def consume(gbuf, sidx, didx, sem):
                            pltpu.make_async_copy(
                                tab.at[sidx], gbuf, sem).wait()
                            pltpu.sync_copy(gbuf, acc.at[didx], add=True)
                            if do_cnt:
                                pltpu.sync_copy(
                                    obuf, cnt_acc.at[didx], add=True)

                        def prefetch(b, gbuf, sidx, didx, sem):
                            off = ebase + b * EB
                            pltpu.sync_copy(src_r.at[pl.ds(off, EB)], sidx)
                            pltpu.sync_copy(dst_r.at[pl.ds(off, EB)], didx)
                            pltpu.async_copy(tab.at[sidx], gbuf, sem)

                        def pair(b2, carry):
                            prefetch(2 * b2 + 1, gbuf1, sidx1, didx1, sem1)
                            consume(gbuf0, sidx0, didx0, sem0)

                            @pl.when(b2 + 1 < NBATCH // 2)
                            def _():
                                prefetch(2 * b2 + 2, gbuf0, sidx0, didx0,
                                         sem0)
                            consume(gbuf1, sidx1, didx1, sem1)
                            return carry

                        lax.fori_loop(0, NBATCH // 2, pair, 0)
                        plsc.subcore_barrier()

                        def flush(flen):
                            def go():
                                pltpu.sync_copy(
                                    acc.at[pl.ds(base, flen), :],
                                    out.at[pl.ds(base, flen), :])
                                if do_cnt:
                                    pltpu.sync_copy(
                                        cnt_acc.at[pl.ds(base, flen)],
                                        cnt_outs[r].at[pl.ds(base, flen)])
                            return go

                        pl.when(sid < NSUB - 1)(flush(STRIPE))
                        pl.when(sid == NSUB - 1)(flush(LAST_FLUSH))
                        plsc.subcore_barrier()

                    pl.when(cid == half)(do_pass)

    return pl.kernel(body, out_type=tuple(out_type), mesh=mesh,
                     scratch_types=scratch,
                     compiler_params=pltpu.CompilerParams(
                         use_tc_tiling_on_sc=False))


_AGG_L1 = _make_agg(2, True, _L1_TMAP)
_AGG_L2 = _make_agg(1, False, _L2_TMAP)

_ROWS = 1000
_GRID = N // _ROWS


def _tc1_body(a0, a1, a2, a3, g0, g1, g2, g3, ca, cb, W1a, W1b, b1,
              W2x, W2y, tx0, tx1, ty0, ty1):
    ia = 1.0 / jnp.maximum(ca[...], 1.0)
    ib = 1.0 / jnp.maximum(cb[...], 1.0)
    h = jnp.broadcast_to(b1[...], (_ROWS, D_HID))
    for k, a in enumerate((a0, a1, a2, a3)):
        h = h + jnp.dot(a[...] * ia, W1a[k * CB:(k + 1) * CB, :],
                        preferred_element_type=f32)
    for k, g in enumerate((g0, g1, g2, g3)):
        h = h + jnp.dot(g[...] * ib, W1b[k * CB:(k + 1) * CB, :],
                        preferred_element_type=f32)
    h = jnp.maximum(h, 0.0)
    tx0[...] = jnp.dot(h, W2x[:, 0:CB], preferred_element_type=f32)
    tx1[...] = jnp.dot(h, W2x[:, CB:2 * CB], preferred_element_type=f32)
    ty0[...] = jnp.dot(h, W2y[:, 0:CB], preferred_element_type=f32)
    ty1[...] = jnp.dot(h, W2y[:, CB:2 * CB], preferred_element_type=f32)


def _tc1(Aa, Ab, ca, cb, W1a, W1b, b1, W2x, W2y):
    blk = lambda i: (i, 0)
    full = lambda i: (0, 0)
    spec_a = pl.BlockSpec((_ROWS, CB), blk)
    spec_c = pl.BlockSpec((_ROWS, 1), blk)
    return pl.pallas_call(
        _tc1_body,
        grid=(_GRID,),
        in_specs=[spec_a] * 8 + [spec_c] * 2 + [
            pl.BlockSpec((D_IN, D_HID), full),
            pl.BlockSpec((D_IN, D_HID), full),
            pl.BlockSpec((1, D_HID), full),
            pl.BlockSpec((D_HID, D_EMB), full),
            pl.BlockSpec((D_HID, D_EMB), full),
        ],
        out_specs=[spec_a] * 4,
        out_shape=[jax.ShapeDtypeStruct((N, CB), f32) for _ in range(4)],
    )(*Aa, *Ab, ca, cb, W1a, W1b, b1, W2x, W2y)


def _tc2_body(x0, x1, g0, g1, ca, cb, b2, out):
    ia = 1.0 / jnp.maximum(ca[...], 1.0)
    ib = 1.0 / jnp.maximum(cb[...], 1.0)
    out[...] = jnp.concatenate(
        [x0[...] * ia + g0[...] * ib, x1[...] * ia + g1[...] * ib],
        axis=1) + b2[...]


def _tc2(Ax, Ag, ca, cb, b2):
    blk = lambda i: (i, 0)
    full = lambda i: (0, 0)
    spec_a = pl.BlockSpec((_ROWS, CB), blk)
    spec_c = pl.BlockSpec((_ROWS, 1), blk)
    return pl.pallas_call(
        _tc2_body,
        grid=(_GRID,),
        in_specs=[spec_a] * 4 + [spec_c] * 2 + [pl.BlockSpec((1, D_EMB), full)],
        out_specs=pl.BlockSpec((_ROWS, D_EMB), blk),
        out_shape=jax.ShapeDtypeStruct((N, D_EMB), f32),
    )(*Ax, *Ag, ca, cb, b2)


def kernel(x_drug, x_gene, src_dd, dst_dd, src_dg, dst_dg, src_gd, dst_gd,
           src_gg, dst_gg, W1_dd, W1_dg, W1_gd, W1_gg, b1_drug, b1_gene,
           W2_dd, W2_dg, W2_gd, W2_gg, b2_drug, b2_gene):
    pad = EP - E
    spad = jnp.zeros((pad,), i32)
    dpad = jnp.full((pad,), DUMP, i32)
    edges = []
    for s, d in ((src_dd, dst_dd), (src_dg, dst_dg),
                 (src_gd, dst_gd), (src_gg, dst_gg)):
        edges.append(jnp.concatenate([s.astype(i32), spad]))
        edges.append(jnp.concatenate([d.astype(i32), dpad]))

    xblocks = [x_drug[:, k * CB:(k + 1) * CB] for k in range(4)]
    xblocks += [x_gene[:, k * CB:(k + 1) * CB] for k in range(4)]

    zc = jnp.zeros((EB, CB), f32)
    oc = jnp.ones((EB,), f32)
    zcn = jnp.zeros((STRIPE,), f32)

    outs = _AGG_L1(*xblocks, *edges, zc, oc, zcn)
    acc1 = [outs[4 * r:4 * (r + 1)] for r in range(4)]
    cnts = [c.reshape(N, 1) for c in outs[16:20]]

    t2_dd0, t2_dd1, t2_dg0, t2_dg1 = _tc1(
        acc1[0], acc1[2], cnts[0], cnts[2], W1_dd, W1_gd,
        b1_drug.reshape(1, D_HID), W2_dd, W2_dg)
    t2_gd0, t2_gd1, t2_gg0, t2_gg1 = _tc1(
        acc1[1], acc1[3], cnts[1], cnts[3], W1_dg, W1_gg,
        b1_gene.reshape(1, D_HID), W2_gd, W2_gg)

    outs2 = _AGG_L2(t2_dd0, t2_dd1, t2_dg0, t2_dg1,
                    t2_gd0, t2_gd1, t2_gg0, t2_gg1, *edges, zc)
    acc2 = [outs2[2 * r:2 * (r + 1)] for r in range(4)]

    o_drug = _tc2(acc2[0], acc2[2], cnts[0], cnts[2],
                  b2_drug.reshape(1, D_EMB))
    o_gene = _tc2(acc2[1], acc2[3], cnts[1], cnts[3],
                  b2_gene.reshape(1, D_EMB))
    return (o_drug, o_gene)


# speedup vs baseline: 3.6007x; 1.2269x over previous
import functools

import jax
import jax.numpy as jnp
from jax import lax
from jax.experimental import pallas as pl
from jax.experimental.pallas import tpu as pltpu
from jax.experimental.pallas import tpu_sc as plsc

N = 50000
E = 150000
D_IN = 128
D_HID = 128
D_EMB = 64
CB = 32

NSUB = 16
NCORE = 2
NPAD = 50048
STRIPE = NPAD // NSUB
LAST_FLUSH = N - (NSUB - 1) * STRIPE
DUMP = N
EB = 320
NBATCH = 30
SLAB = EB * NBATCH
EP = SLAB * NSUB

f32 = jnp.float32
i32 = jnp.int32

_L1_TMAP = ((0, 1), (0, 1), (2, 3), (2, 3))
_L2_TMAP = ((0, 1), (2, 3), (4, 5), (6, 7))


def _make_agg(nblk_per_core, with_counts, tmap, ntab, cb, dt, eb):
    nblk = nblk_per_core * NCORE
    nbatch = SLAB // eb
    mesh = plsc.VectorSubcoreMesh(core_axis_name="c", subcore_axis_name="s",
                                  num_cores=NCORE, num_subcores=NSUB)
    out_type = [jax.ShapeDtypeStruct((N, cb), dt) for _ in range(4 * nblk)]
    if with_counts:
        out_type += [jax.ShapeDtypeStruct((N,), f32) for _ in range(4)]
    scratch = [
        pltpu.VMEM_SHARED((NPAD, cb), dt),
        pltpu.VMEM((eb, cb), dt),
        pltpu.VMEM((eb, cb), dt),
        pltpu.VMEM((eb,), i32),
        pltpu.VMEM((eb,), i32),
        pltpu.VMEM((eb,), i32),
        pltpu.VMEM((eb,), i32),
        pltpu.SemaphoreType.DMA,
        pltpu.SemaphoreType.DMA,
    ]
    if with_counts:
        scratch += [
            pltpu.VMEM_SHARED((NPAD,), f32),
            pltpu.VMEM((eb,), f32),
            pltpu.VMEM((STRIPE,), f32),
        ]

    def body(*refs):
        tabs = refs[0:ntab]
        edges = refs[ntab:ntab + 8]
        zc = refs[ntab + 8]
        k = ntab + 9
        if with_counts:
            oc, zcn = refs[k], refs[k + 1]
            k += 2
        outs = refs[k:k + 4 * nblk]
        k += 4 * nblk
        if with_counts:
            cnt_outs = refs[k:k + 4]
            k += 4
        acc, gbuf0, gbuf1, sidx0, sidx1, didx0, didx1, sem0, sem1 = \
            refs[k:k + 9]
        if with_counts:
            cnt_acc, obuf, z1buf = refs[k + 9:k + 12]

        cid = lax.axis_index("c")
        sid = lax.axis_index("s")
        if with_counts:
            pltpu.sync_copy(oc, obuf)
            pltpu.sync_copy(zcn, z1buf)
        base = sid * STRIPE
        ebase = sid * SLAB

        for r in range(4):
            src_r = edges[2 * r]
            dst_r = edges[2 * r + 1]
            for p in range(nblk_per_core):
                for half in range(NCORE):
                    blk = half * nblk_per_core + p
                    tab = tabs[tmap[r][blk]]
                    out = outs[r * nblk + blk]
                    do_cnt = with_counts and blk == 0

                    def do_pass(tab=tab, out=out, src_r=src_r, dst_r=dst_r,
                                do_cnt=do_cnt, r=r):
                        pltpu.sync_copy(zc, gbuf0)
                        for z in range(STRIPE // eb):
                            pltpu.sync_copy(
                                gbuf0, acc.at[pl.ds(base + z * eb, eb), :])
                        rem = STRIPE % eb
                        if rem:
                            pltpu.sync_copy(
                                gbuf0.at[pl.ds(0, rem), :],
                                acc.at[pl.ds(base + STRIPE - rem, rem), :])
                        if do_cnt:
                            pltpu.sync_copy(
                                z1buf, cnt_acc.at[pl.ds(base, STRIPE)])
                        plsc.subcore_barrier()

                        pltpu.sync_copy(src_r.at[pl.ds(ebase, eb)], sidx0)
                        pltpu.sync_copy(dst_r.at[pl.ds(ebase, eb)], didx0)
                        pltpu.async_copy(tab.at[sidx0], gbuf0, sem0)

                        def consume(gbuf, sidx, didx, sem):
                            pltpu.make_async_copy(
                                tab.at[sidx], gbuf, sem).wait()
                            pltpu.sync_copy(gbuf, acc.at[didx], add=True)
                            if do_cnt:
                                pltpu.sync_copy(
                                    obuf, cnt_acc.at[didx], add=True)

                        def prefetch(b, gbuf, sidx, didx, sem):
                            off = ebase + b * eb
                            pltpu.sync_copy(src_r.at[pl.ds(off, eb)], sidx)
                            pltpu.sync_copy(dst_r.at[pl.ds(off, eb)], didx)
                            pltpu.async_copy(tab.at[sidx], gbuf, sem)

                        def pair(b2, carry):
                            prefetch(2 * b2 + 1, gbuf1, sidx1, didx1, sem1)
                            consume(gbuf0, sidx0, didx0, sem0)

                            @pl.when(b2 + 1 < nbatch // 2)
                            def _():
                                prefetch(2 * b2 + 2, gbuf0, sidx0, didx0,
                                         sem0)
                            consume(gbuf1, sidx1, didx1, sem1)
                            return carry

                        lax.fori_loop(0, nbatch // 2, pair, 0)
                        plsc.subcore_barrier()

                        def flush(flen):
                            def go():
                                pltpu.sync_copy(
                                    acc.at[pl.ds(base, flen), :],
                                    out.at[pl.ds(base, flen), :])
                                if do_cnt:
                                    pltpu.sync_copy(
                                        cnt_acc.at[pl.ds(base, flen)],
                                        cnt_outs[r].at[pl.ds(base, flen)])
                            return go

                        pl.when(sid < NSUB - 1)(flush(STRIPE))
                        pl.when(sid == NSUB - 1)(flush(LAST_FLUSH))
                        plsc.subcore_barrier()

                    pl.when(cid == half)(do_pass)

    return pl.kernel(body, out_type=tuple(out_type), mesh=mesh,
                     scratch_types=scratch,
                     compiler_params=pltpu.CompilerParams(
                         use_tc_tiling_on_sc=False))


HB = 64
bf16 = jnp.bfloat16
_AGG_L1 = _make_agg(1, True, _L1_TMAP, 4, HB, bf16, 320)
_AGG_L2 = _make_agg(1, False, _L2_TMAP, 8, CB, f32, 320)

_ROWS = 1000
_GRID = N // _ROWS


def _tc1_body(a0, a1, g0, g1, ca, cb, W1a, W1b, b1,
              W2x, W2y, tx0, tx1, ty0, ty1):
    ia = 1.0 / jnp.maximum(ca[...], 1.0)
    ib = 1.0 / jnp.maximum(cb[...], 1.0)
    h = jnp.broadcast_to(b1[...], (_ROWS, D_HID))
    for k, a in enumerate((a0, a1)):
        h = h + jnp.dot(a[...].astype(f32) * ia, W1a[k * HB:(k + 1) * HB, :],
                        preferred_element_type=f32)
    for k, g in enumerate((g0, g1)):
        h = h + jnp.dot(g[...].astype(f32) * ib, W1b[k * HB:(k + 1) * HB, :],
                        preferred_element_type=f32)
    h = jnp.maximum(h, 0.0)
    tx0[...] = jnp.dot(h, W2x[:, 0:CB], preferred_element_type=f32)
    tx1[...] = jnp.dot(h, W2x[:, CB:2 * CB], preferred_element_type=f32)
    ty0[...] = jnp.dot(h, W2y[:, 0:CB], preferred_element_type=f32)
    ty1[...] = jnp.dot(h, W2y[:, CB:2 * CB], preferred_element_type=f32)


def _tc1(Aa, Ab, ca, cb, W1a, W1b, b1, W2x, W2y):
    blk = lambda i: (i, 0)
    full = lambda i: (0, 0)
    spec_a = pl.BlockSpec((_ROWS, HB), blk)
    spec_c = pl.BlockSpec((_ROWS, 1), blk)
    return pl.pallas_call(
        _tc1_body,
        grid=(_GRID,),
        in_specs=[spec_a] * 4 + [spec_c] * 2 + [
            pl.BlockSpec((D_IN, D_HID), full),
            pl.BlockSpec((D_IN, D_HID), full),
            pl.BlockSpec((1, D_HID), full),
            pl.BlockSpec((D_HID, D_EMB), full),
            pl.BlockSpec((D_HID, D_EMB), full),
        ],
        out_specs=[pl.BlockSpec((_ROWS, CB), blk)] * 4,
        out_shape=[jax.ShapeDtypeStruct((N, CB), f32) for _ in range(4)],
    )(*Aa, *Ab, ca, cb, W1a, W1b, b1, W2x, W2y)


def _tc2_body(x0, x1, g0, g1, ca, cb, b2, out):
    ia = 1.0 / jnp.maximum(ca[...], 1.0)
    ib = 1.0 / jnp.maximum(cb[...], 1.0)
    out[...] = jnp.concatenate(
        [x0[...] * ia + g0[...] * ib, x1[...] * ia + g1[...] * ib],
        axis=1) + b2[...]


def _tc2(Ax, Ag, ca, cb, b2):
    blk = lambda i: (i, 0)
    full = lambda i: (0, 0)
    spec_a = pl.BlockSpec((_ROWS, CB), blk)
    spec_c = pl.BlockSpec((_ROWS, 1), blk)
    return pl.pallas_call(
        _tc2_body,
        grid=(_GRID,),
        in_specs=[spec_a] * 4 + [spec_c] * 2 + [pl.BlockSpec((1, D_EMB), full)],
        out_specs=pl.BlockSpec((_ROWS, D_EMB), blk),
        out_shape=jax.ShapeDtypeStruct((N, D_EMB), f32),
    )(*Ax, *Ag, ca, cb, b2)


def kernel(x_drug, x_gene, src_dd, dst_dd, src_dg, dst_dg, src_gd, dst_gd,
           src_gg, dst_gg, W1_dd, W1_dg, W1_gd, W1_gg, b1_drug, b1_gene,
           W2_dd, W2_dg, W2_gd, W2_gg, b2_drug, b2_gene):
    pad = EP - E
    spad = jnp.zeros((pad,), i32)
    dpad = jnp.full((pad,), DUMP, i32)
    edges = []
    for s, d in ((src_dd, dst_dd), (src_dg, dst_dg),
                 (src_gd, dst_gd), (src_gg, dst_gg)):
        edges.append(jnp.concatenate([s.astype(i32), spad]))
        edges.append(jnp.concatenate([d.astype(i32), dpad]))

    xd16 = x_drug.astype(bf16)
    xg16 = x_gene.astype(bf16)
    xblocks = [xd16[:, :HB], xd16[:, HB:], xg16[:, :HB], xg16[:, HB:]]

    zc1 = jnp.zeros((320, HB), bf16)
    zc2 = jnp.zeros((320, CB), f32)
    oc = jnp.ones((320,), f32)
    zcn = jnp.zeros((STRIPE,), f32)

    outs = _AGG_L1(*xblocks, *edges, zc1, oc, zcn)
    acc1 = [outs[2 * r:2 * (r + 1)] for r in range(4)]
    cnts = [c.reshape(N, 1) for c in outs[8:12]]

    t2_dd0, t2_dd1, t2_dg0, t2_dg1 = _tc1(
        acc1[0], acc1[2], cnts[0], cnts[2], W1_dd, W1_gd,
        b1_drug.reshape(1, D_HID), W2_dd, W2_dg)
    t2_gd0, t2_gd1, t2_gg0, t2_gg1 = _tc1(
        acc1[1], acc1[3], cnts[1], cnts[3], W1_dg, W1_gg,
        b1_gene.reshape(1, D_HID), W2_gd, W2_gg)

    outs2 = _AGG_L2(t2_dd0, t2_dd1, t2_dg0, t2_dg1,
                    t2_gd0, t2_gd1, t2_gg0, t2_gg1, *edges, zc2)
    acc2 = [outs2[2 * r:2 * (r + 1)] for r in range(4)]

    o_drug = _tc2(acc2[0], acc2[2], cnts[0], cnts[2],
                  b2_drug.reshape(1, D_EMB))
    o_gene = _tc2(acc2[1], acc2[3], cnts[1], cnts[3],
                  b2_gene.reshape(1, D_EMB))
    return (o_drug, o_gene)


# speedup vs baseline: 3.7103x; 1.0304x over previous
import functools

import jax
import jax.numpy as jnp
from jax import lax
from jax.experimental import pallas as pl
from jax.experimental.pallas import tpu as pltpu
from jax.experimental.pallas import tpu_sc as plsc

N = 50000
E = 150000
D_IN = 128
D_HID = 128
D_EMB = 64
CB = 32

NSUB = 16
NCORE = 2
NPAD = 50048
STRIPE = NPAD // NSUB
LAST_FLUSH = N - (NSUB - 1) * STRIPE
DUMP = N
EB = 320
NBATCH = 30
SLAB = EB * NBATCH
EP = SLAB * NSUB

f32 = jnp.float32
i32 = jnp.int32

_L1_TMAP = ((0, 1), (0, 1), (2, 3), (2, 3))
_L2_TMAP = ((0, 1), (2, 3), (4, 5), (6, 7))


def _make_agg(nblk_per_core, with_counts, tmap, ntab, cb, dt, eb):
    nblk = nblk_per_core * NCORE
    nbatch = SLAB // eb
    mesh = plsc.VectorSubcoreMesh(core_axis_name="c", subcore_axis_name="s",
                                  num_cores=NCORE, num_subcores=NSUB)
    out_type = [jax.ShapeDtypeStruct((N, cb), dt) for _ in range(4 * nblk)]
    if with_counts:
        out_type += [jax.ShapeDtypeStruct((N,), f32) for _ in range(4)]
    scratch = [
        pltpu.VMEM_SHARED((NPAD, cb), dt),
        pltpu.VMEM((eb, cb), dt),
        pltpu.VMEM((eb, cb), dt),
        pltpu.VMEM((eb,), i32),
        pltpu.VMEM((eb,), i32),
        pltpu.VMEM((eb,), i32),
        pltpu.VMEM((eb,), i32),
        pltpu.SemaphoreType.DMA,
        pltpu.SemaphoreType.DMA,
    ]
    if with_counts:
        scratch += [
            pltpu.VMEM_SHARED((NPAD,), f32),
            pltpu.VMEM((eb,), f32),
            pltpu.VMEM((STRIPE,), f32),
        ]

    def body(*refs):
        tabs = refs[0:ntab]
        edges = refs[ntab:ntab + 8]
        zc = refs[ntab + 8]
        k = ntab + 9
        if with_counts:
            oc, zcn = refs[k], refs[k + 1]
            k += 2
        outs = refs[k:k + 4 * nblk]
        k += 4 * nblk
        if with_counts:
            cnt_outs = refs[k:k + 4]
            k += 4
        acc, gbuf0, gbuf1, sidx0, sidx1, didx0, didx1, sem0, sem1 = \
            refs[k:k + 9]
        if with_counts:
            cnt_acc, obuf, z1buf = refs[k + 9:k + 12]

        cid = lax.axis_index("c")
        sid = lax.axis_index("s")
        if with_counts:
            pltpu.sync_copy(oc, obuf)
            pltpu.sync_copy(zcn, z1buf)
        base = sid * STRIPE
        ebase = sid * SLAB

        for r in range(4):
            src_r = edges[2 * r]
            dst_r = edges[2 * r + 1]
            for p in range(nblk_per_core):
                for half in range(NCORE):
                    blk = half * nblk_per_core + p
                    tab = tabs[tmap[r][blk]]
                    out = outs[r * nblk + blk]
                    do_cnt = with_counts and blk == 0

                    def do_pass(tab=tab, out=out, src_r=src_r, dst_r=dst_r,
                                do_cnt=do_cnt, r=r):
                        pltpu.sync_copy(zc, gbuf0)
                        for z in range(STRIPE // eb):
                            pltpu.sync_copy(
                                gbuf0, acc.at[pl.ds(base + z * eb, eb), :])
                        rem = STRIPE % eb
                        if rem:
                            pltpu.sync_copy(
                                gbuf0.at[pl.ds(0, rem), :],
                                acc.at[pl.ds(base + STRIPE - rem, rem), :])
                        if do_cnt:
                            pltpu.sync_copy(
                                z1buf, cnt_acc.at[pl.ds(base, STRIPE)])
                        plsc.subcore_barrier()

                        pltpu.sync_copy(src_r.at[pl.ds(ebase, eb)], sidx0)
                        pltpu.sync_copy(dst_r.at[pl.ds(ebase, eb)], didx0)
                        pltpu.async_copy(tab.at[sidx0], gbuf0, sem0)

                        def consume(gbuf, sidx, didx, sem):
                            pltpu.make_async_copy(
                                tab.at[sidx], gbuf, sem).wait()
                            pltpu.sync_copy(gbuf, acc.at[didx], add=True)
                            if do_cnt:
                                pltpu.sync_copy(
                                    obuf, cnt_acc.at[didx], add=True)

                        def prefetch(b, gbuf, sidx, didx, sem):
                            off = ebase + b * eb
                            pltpu.sync_copy(src_r.at[pl.ds(off, eb)], sidx)
                            pltpu.sync_copy(dst_r.at[pl.ds(off, eb)], didx)
                            pltpu.async_copy(tab.at[sidx], gbuf, sem)

                        def pair(b2, carry):
                            prefetch(2 * b2 + 1, gbuf1, sidx1, didx1, sem1)
                            consume(gbuf0, sidx0, didx0, sem0)

                            @pl.when(b2 + 1 < nbatch // 2)
                            def _():
                                prefetch(2 * b2 + 2, gbuf0, sidx0, didx0,
                                         sem0)
                            consume(gbuf1, sidx1, didx1, sem1)
                            return carry

                        lax.fori_loop(0, nbatch // 2, pair, 0)
                        plsc.subcore_barrier()

                        def flush(flen):
                            def go():
                                pltpu.sync_copy(
                                    acc.at[pl.ds(base, flen), :],
                                    out.at[pl.ds(base, flen), :])
                                if do_cnt:
                                    pltpu.sync_copy(
                                        cnt_acc.at[pl.ds(base, flen)],
                                        cnt_outs[r].at[pl.ds(base, flen)])
                            return go

                        pl.when(sid < NSUB - 1)(flush(STRIPE))
                        pl.when(sid == NSUB - 1)(flush(LAST_FLUSH))
                        plsc.subcore_barrier()

                    pl.when(cid == half)(do_pass)

    return pl.kernel(body, out_type=tuple(out_type), mesh=mesh,
                     scratch_types=scratch,
                     compiler_params=pltpu.CompilerParams(
                         use_tc_tiling_on_sc=False))


HB = 64
bf16 = jnp.bfloat16
_AGG_L1 = _make_agg(1, True, _L1_TMAP, 4, HB, bf16, 320)
_AGG_L2 = _make_agg(1, False, _L2_TMAP, 8, CB, bf16, 600)

_ROWS = 1000
_GRID = N // _ROWS


def _tc1_body(a0, a1, g0, g1, ca, cb, W1a, W1b, b1,
              W2x, W2y, tx0, tx1, ty0, ty1):
    ia = 1.0 / jnp.maximum(ca[...], 1.0)
    ib = 1.0 / jnp.maximum(cb[...], 1.0)
    h = jnp.broadcast_to(b1[...], (_ROWS, D_HID))
    for k, a in enumerate((a0, a1)):
        h = h + jnp.dot(a[...].astype(f32) * ia, W1a[k * HB:(k + 1) * HB, :],
                        preferred_element_type=f32)
    for k, g in enumerate((g0, g1)):
        h = h + jnp.dot(g[...].astype(f32) * ib, W1b[k * HB:(k + 1) * HB, :],
                        preferred_element_type=f32)
    h = jnp.maximum(h, 0.0)
    tx0[...] = jnp.dot(h, W2x[:, 0:CB],
                       preferred_element_type=f32).astype(bf16)
    tx1[...] = jnp.dot(h, W2x[:, CB:2 * CB],
                       preferred_element_type=f32).astype(bf16)
    ty0[...] = jnp.dot(h, W2y[:, 0:CB],
                       preferred_element_type=f32).astype(bf16)
    ty1[...] = jnp.dot(h, W2y[:, CB:2 * CB],
                       preferred_element_type=f32).astype(bf16)


def _tc1(Aa, Ab, ca, cb, W1a, W1b, b1, W2x, W2y):
    blk = lambda i: (i, 0)
    full = lambda i: (0, 0)
    spec_a = pl.BlockSpec((_ROWS, HB), blk)
    spec_c = pl.BlockSpec((_ROWS, 1), blk)
    return pl.pallas_call(
        _tc1_body,
        grid=(_GRID,),
        in_specs=[spec_a] * 4 + [spec_c] * 2 + [
            pl.BlockSpec((D_IN, D_HID), full),
            pl.BlockSpec((D_IN, D_HID), full),
            pl.BlockSpec((1, D_HID), full),
            pl.BlockSpec((D_HID, D_EMB), full),
            pl.BlockSpec((D_HID, D_EMB), full),
        ],
        out_specs=[pl.BlockSpec((_ROWS, CB), blk)] * 4,
        out_shape=[jax.ShapeDtypeStruct((N, CB), bf16) for _ in range(4)],
    )(*Aa, *Ab, ca, cb, W1a, W1b, b1, W2x, W2y)


def _tc2_body(x0, x1, g0, g1, ca, cb, b2, out):
    ia = 1.0 / jnp.maximum(ca[...], 1.0)
    ib = 1.0 / jnp.maximum(cb[...], 1.0)
    out[...] = jnp.concatenate(
        [x0[...].astype(f32) * ia + g0[...].astype(f32) * ib,
         x1[...].astype(f32) * ia + g1[...].astype(f32) * ib],
        axis=1) + b2[...]


def _tc2(Ax, Ag, ca, cb, b2):
    blk = lambda i: (i, 0)
    full = lambda i: (0, 0)
    spec_a = pl.BlockSpec((_ROWS, CB), blk)
    spec_c = pl.BlockSpec((_ROWS, 1), blk)
    return pl.pallas_call(
        _tc2_body,
        grid=(_GRID,),
        in_specs=[spec_a] * 4 + [spec_c] * 2 + [pl.BlockSpec((1, D_EMB), full)],
        out_specs=pl.BlockSpec((_ROWS, D_EMB), blk),
        out_shape=jax.ShapeDtypeStruct((N, D_EMB), f32),
    )(*Ax, *Ag, ca, cb, b2)


def kernel(x_drug, x_gene, src_dd, dst_dd, src_dg, dst_dg, src_gd, dst_gd,
           src_gg, dst_gg, W1_dd, W1_dg, W1_gd, W1_gg, b1_drug, b1_gene,
           W2_dd, W2_dg, W2_gd, W2_gg, b2_drug, b2_gene):
    pad = EP - E
    spad = jnp.zeros((pad,), i32)
    dpad = jnp.full((pad,), DUMP, i32)
    edges = []
    for s, d in ((src_dd, dst_dd), (src_dg, dst_dg),
                 (src_gd, dst_gd), (src_gg, dst_gg)):
        edges.append(jnp.concatenate([s.astype(i32), spad]))
        edges.append(jnp.concatenate([d.astype(i32), dpad]))

    xd16 = x_drug.astype(bf16)
    xg16 = x_gene.astype(bf16)
    xblocks = [xd16[:, :HB], xd16[:, HB:], xg16[:, :HB], xg16[:, HB:]]

    zc1 = jnp.zeros((320, HB), bf16)
    zc2 = jnp.zeros((600, CB), bf16)
    oc = jnp.ones((320,), f32)
    zcn = jnp.zeros((STRIPE,), f32)

    outs = _AGG_L1(*xblocks, *edges, zc1, oc, zcn)
    acc1 = [outs[2 * r:2 * (r + 1)] for r in range(4)]
    cnts = [c.reshape(N, 1) for c in outs[8:12]]

    t2_dd0, t2_dd1, t2_dg0, t2_dg1 = _tc1(
        acc1[0], acc1[2], cnts[0], cnts[2], W1_dd, W1_gd,
        b1_drug.reshape(1, D_HID), W2_dd, W2_dg)
    t2_gd0, t2_gd1, t2_gg0, t2_gg1 = _tc1(
        acc1[1], acc1[3], cnts[1], cnts[3], W1_dg, W1_gg,
        b1_gene.reshape(1, D_HID), W2_gd, W2_gg)

    outs2 = _AGG_L2(t2_dd0, t2_dd1, t2_dg0, t2_dg1,
                    t2_gd0, t2_gd1, t2_gg0, t2_gg1, *edges, zc2)
    acc2 = [outs2[2 * r:2 * (r + 1)] for r in range(4)]

    o_drug = _tc2(acc2[0], acc2[2], cnts[0], cnts[2],
                  b2_drug.reshape(1, D_EMB))
    o_gene = _tc2(acc2[1], acc2[3], cnts[1], cnts[3],
                  b2_gene.reshape(1, D_EMB))
    return (o_drug, o_gene)


# speedup vs baseline: 3.7243x; 1.0038x over previous
import functools

import jax
import jax.numpy as jnp
from jax import lax
from jax.experimental import pallas as pl
from jax.experimental.pallas import tpu as pltpu
from jax.experimental.pallas import tpu_sc as plsc

N = 50000
E = 150000
D_IN = 128
D_HID = 128
D_EMB = 64
CB = 32

NSUB = 16
NCORE = 2
NPAD = 50048
STRIPE = NPAD // NSUB
LAST_FLUSH = N - (NSUB - 1) * STRIPE
DUMP = N
EB = 320
NBATCH = 30
SLAB = EB * NBATCH
EP = SLAB * NSUB

f32 = jnp.float32
i32 = jnp.int32

_L1_TMAP = ((0, 1), (0, 1), (2, 3), (2, 3))
_L2_TMAP = ((0, 1), (2, 3), (4, 5), (6, 7))


def _make_agg(nblk_per_core, with_counts, tmap, ntab, cb, dt, eb):
    nblk = nblk_per_core * NCORE
    nbatch = SLAB // eb
    mesh = plsc.VectorSubcoreMesh(core_axis_name="c", subcore_axis_name="s",
                                  num_cores=NCORE, num_subcores=NSUB)
    out_type = [jax.ShapeDtypeStruct((N, cb), dt) for _ in range(4 * nblk)]
    if with_counts:
        out_type += [jax.ShapeDtypeStruct((4, N), f32)]
    scratch = [
        pltpu.VMEM_SHARED((NPAD, cb), dt),
        pltpu.VMEM((eb, cb), dt),
        pltpu.VMEM((eb, cb), dt),
        pltpu.VMEM((eb,), i32),
        pltpu.VMEM((eb,), i32),
        pltpu.VMEM((eb,), i32),
        pltpu.VMEM((eb,), i32),
        pltpu.SemaphoreType.DMA,
        pltpu.SemaphoreType.DMA,
    ]
    if with_counts:
        scratch += [
            pltpu.VMEM_SHARED((NPAD,), f32),
            pltpu.VMEM((eb,), f32),
            pltpu.VMEM((STRIPE,), f32),
        ]

    def body(*refs):
        tabs = refs[0:ntab]
        edges = refs[ntab]
        zc = refs[ntab + 1]
        k = ntab + 2
        if with_counts:
            oc, zcn = refs[k], refs[k + 1]
            k += 2
        outs = refs[k:k + 4 * nblk]
        k += 4 * nblk
        if with_counts:
            cnt_out = refs[k]
            k += 1
        acc, gbuf0, gbuf1, sidx0, sidx1, didx0, didx1, sem0, sem1 = \
            refs[k:k + 9]
        if with_counts:
            cnt_acc, obuf, z1buf = refs[k + 9:k + 12]

        cid = lax.axis_index("c")
        sid = lax.axis_index("s")
        if with_counts:
            pltpu.sync_copy(oc, obuf)
            pltpu.sync_copy(zcn, z1buf)
        base = sid * STRIPE
        ebase = sid * SLAB

        for r in range(4):
            for p in range(nblk_per_core):
                for half in range(NCORE):
                    blk = half * nblk_per_core + p
                    tab = tabs[tmap[r][blk]]
                    out = outs[r * nblk + blk]
                    do_cnt = with_counts and blk == 0

                    def do_pass(tab=tab, out=out, do_cnt=do_cnt, r=r):
                        pltpu.sync_copy(zc, gbuf0)
                        for z in range(STRIPE // eb):
                            pltpu.sync_copy(
                                gbuf0, acc.at[pl.ds(base + z * eb, eb), :])
                        rem = STRIPE % eb
                        if rem:
                            pltpu.sync_copy(
                                gbuf0.at[pl.ds(0, rem), :],
                                acc.at[pl.ds(base + STRIPE - rem, rem), :])
                        if do_cnt:
                            pltpu.sync_copy(
                                z1buf, cnt_acc.at[pl.ds(base, STRIPE)])
                        plsc.subcore_barrier()

                        pltpu.sync_copy(
                            edges.at[2 * r, pl.ds(ebase, eb)], sidx0)
                        pltpu.sync_copy(
                            edges.at[2 * r + 1, pl.ds(ebase, eb)], didx0)
                        pltpu.async_copy(tab.at[sidx0], gbuf0, sem0)

                        def consume(gbuf, sidx, didx, sem):
                            pltpu.make_async_copy(
                                tab.at[sidx], gbuf, sem).wait()
                            pltpu.sync_copy(gbuf, acc.at[didx], add=True)
                            if do_cnt:
                                pltpu.sync_copy(
                                    obuf, cnt_acc.at[didx], add=True)

                        def prefetch(b, gbuf, sidx, didx, sem):
                            off = ebase + b * eb
                            pltpu.sync_copy(
                                edges.at[2 * r, pl.ds(off, eb)], sidx)
                            pltpu.sync_copy(
                                edges.at[2 * r + 1, pl.ds(off, eb)], didx)
                            pltpu.async_copy(tab.at[sidx], gbuf, sem)

                        def pair(b2, carry):
                            prefetch(2 * b2 + 1, gbuf1, sidx1, didx1, sem1)
                            consume(gbuf0, sidx0, didx0, sem0)

                            @pl.when(b2 + 1 < nbatch // 2)
                            def _():
                                prefetch(2 * b2 + 2, gbuf0, sidx0, didx0,
                                         sem0)
                            consume(gbuf1, sidx1, didx1, sem1)
                            return carry

                        lax.fori_loop(0, nbatch // 2, pair, 0)
                        plsc.subcore_barrier()

                        def flush(flen):
                            def go():
                                pltpu.sync_copy(
                                    acc.at[pl.ds(base, flen), :],
                                    out.at[pl.ds(base, flen), :])
                                if do_cnt:
                                    pltpu.sync_copy(
                                        cnt_acc.at[pl.ds(base, flen)],
                                        cnt_out.at[r, pl.ds(base, flen)])
                            return go

                        pl.when(sid < NSUB - 1)(flush(STRIPE))
                        pl.when(sid == NSUB - 1)(flush(LAST_FLUSH))
                        plsc.subcore_barrier()

                    pl.when(cid == half)(do_pass)

    return pl.kernel(body, out_type=tuple(out_type), mesh=mesh,
                     scratch_types=scratch,
                     compiler_params=pltpu.CompilerParams(
                         use_tc_tiling_on_sc=False))


HB = 64
bf16 = jnp.bfloat16
_AGG_L1 = _make_agg(1, True, _L1_TMAP, 4, HB, bf16, 320)
_AGG_L2 = _make_agg(1, False, _L2_TMAP, 8, CB, bf16, 600)

_ROWS = 1000
_GRID = N // _ROWS


def _tc1_body(a0, a1, g0, g1, ca, cb, W1a, W1b, b1,
              W2x, W2y, tx0, tx1, ty0, ty1):
    ia = 1.0 / jnp.maximum(ca[...], 1.0)
    ib = 1.0 / jnp.maximum(cb[...], 1.0)
    h = jnp.broadcast_to(b1[...], (_ROWS, D_HID))
    for k, a in enumerate((a0, a1)):
        h = h + jnp.dot(a[...].astype(f32) * ia, W1a[k * HB:(k + 1) * HB, :],
                        preferred_element_type=f32)
    for k, g in enumerate((g0, g1)):
        h = h + jnp.dot(g[...].astype(f32) * ib, W1b[k * HB:(k + 1) * HB, :],
                        preferred_element_type=f32)
    h = jnp.maximum(h, 0.0)
    tx0[...] = jnp.dot(h, W2x[:, 0:CB],
                       preferred_element_type=f32).astype(bf16)
    tx1[...] = jnp.dot(h, W2x[:, CB:2 * CB],
                       preferred_element_type=f32).astype(bf16)
    ty0[...] = jnp.dot(h, W2y[:, 0:CB],
                       preferred_element_type=f32).astype(bf16)
    ty1[...] = jnp.dot(h, W2y[:, CB:2 * CB],
                       preferred_element_type=f32).astype(bf16)


def _tc1(Aa, Ab, ca, cb, W1a, W1b, b1, W2x, W2y):
    blk = lambda i: (i, 0)
    full = lambda i: (0, 0)
    spec_a = pl.BlockSpec((_ROWS, HB), blk)
    spec_c = pl.BlockSpec((_ROWS, 1), blk)
    return pl.pallas_call(
        _tc1_body,
        grid=(_GRID,),
        in_specs=[spec_a] * 4 + [spec_c] * 2 + [
            pl.BlockSpec((D_IN, D_HID), full),
            pl.BlockSpec((D_IN, D_HID), full),
            pl.BlockSpec((1, D_HID), full),
            pl.BlockSpec((D_HID, D_EMB), full),
            pl.BlockSpec((D_HID, D_EMB), full),
        ],
        out_specs=[pl.BlockSpec((_ROWS, CB), blk)] * 4,
        out_shape=[jax.ShapeDtypeStruct((N, CB), bf16) for _ in range(4)],
    )(*Aa, *Ab, ca, cb, W1a, W1b, b1, W2x, W2y)


def _tc2_body(x0, x1, g0, g1, ca, cb, b2, out):
    ia = 1.0 / jnp.maximum(ca[...], 1.0)
    ib = 1.0 / jnp.maximum(cb[...], 1.0)
    out[...] = jnp.concatenate(
        [x0[...].astype(f32) * ia + g0[...].astype(f32) * ib,
         x1[...].astype(f32) * ia + g1[...].astype(f32) * ib],
        axis=1) + b2[...]


def _tc2(Ax, Ag, ca, cb, b2):
    blk = lambda i: (i, 0)
    full = lambda i: (0, 0)
    spec_a = pl.BlockSpec((_ROWS, CB), blk)
    spec_c = pl.BlockSpec((_ROWS, 1), blk)
    return pl.pallas_call(
        _tc2_body,
        grid=(_GRID,),
        in_specs=[spec_a] * 4 + [spec_c] * 2 + [pl.BlockSpec((1, D_EMB), full)],
        out_specs=pl.BlockSpec((_ROWS, D_EMB), blk),
        out_shape=jax.ShapeDtypeStruct((N, D_EMB), f32),
    )(*Ax, *Ag, ca, cb, b2)


def kernel(x_drug, x_gene, src_dd, dst_dd, src_dg, dst_dg, src_gd, dst_gd,
           src_gg, dst_gg, W1_dd, W1_dg, W1_gd, W1_gg, b1_drug, b1_gene,
           W2_dd, W2_dg, W2_gd, W2_gg, b2_drug, b2_gene):
    pad = EP - E
    spad = jnp.zeros((pad,), i32)
    dpad = jnp.full((pad,), DUMP, i32)
    erows = []
    for s, d in ((src_dd, dst_dd), (src_dg, dst_dg),
                 (src_gd, dst_gd), (src_gg, dst_gg)):
        erows.append(jnp.concatenate([s.astype(i32), spad]))
        erows.append(jnp.concatenate([d.astype(i32), dpad]))
    edges = jnp.stack(erows)

    xd16 = x_drug.astype(bf16)
    xg16 = x_gene.astype(bf16)
    xblocks = [xd16[:, :HB], xd16[:, HB:], xg16[:, :HB], xg16[:, HB:]]

    zc1 = jnp.zeros((320, HB), bf16)
    zc2 = jnp.zeros((600, CB), bf16)
    oc = jnp.ones((320,), f32)
    zcn = jnp.zeros((STRIPE,), f32)

    outs = _AGG_L1(*xblocks, edges, zc1, oc, zcn)
    acc1 = [outs[2 * r:2 * (r + 1)] for r in range(4)]
    cnts = [outs[8][r].reshape(N, 1) for r in range(4)]

    t2_dd0, t2_dd1, t2_dg0, t2_dg1 = _tc1(
        acc1[0], acc1[2], cnts[0], cnts[2], W1_dd, W1_gd,
        b1_drug.reshape(1, D_HID), W2_dd, W2_dg)
    t2_gd0, t2_gd1, t2_gg0, t2_gg1 = _tc1(
        acc1[1], acc1[3], cnts[1], cnts[3], W1_dg, W1_gg,
        b1_gene.reshape(1, D_HID), W2_gd, W2_gg)

    outs2 = _AGG_L2(t2_dd0, t2_dd1, t2_dg0, t2_dg1,
                    t2_gd0, t2_gd1, t2_gg0, t2_gg1, edges, zc2)
    acc2 = [outs2[2 * r:2 * (r + 1)] for r in range(4)]

    o_drug = _tc2(acc2[0], acc2[2], cnts[0], cnts[2],
                  b2_drug.reshape(1, D_EMB))
    o_gene = _tc2(acc2[1], acc2[3], cnts[1], cnts[3],
                  b2_gene.reshape(1, D_EMB))
    return (o_drug, o_gene)


# speedup vs baseline: 3.7462x; 1.0059x over previous
import functools

import jax
import jax.numpy as jnp
from jax import lax
from jax.experimental import pallas as pl
from jax.experimental.pallas import tpu as pltpu
from jax.experimental.pallas import tpu_sc as plsc

N = 50000
E = 150000
D_IN = 128
D_HID = 128
D_EMB = 64
CB = 32

NSUB = 16
NCORE = 2
NPAD = 50048
STRIPE = NPAD // NSUB
LAST_FLUSH = N - (NSUB - 1) * STRIPE
DUMP = N
EB = 320
NBATCH = 30
SLAB = EB * NBATCH
EP = SLAB * NSUB

f32 = jnp.float32
i32 = jnp.int32

_L1_TMAP = ((0, 1), (0, 1), (2, 3), (2, 3))
_L2_TMAP = ((0, 1), (2, 3), (4, 5), (6, 7))


def _make_agg(nblk_per_core, with_counts, tmap, ntab, cb, dt, eb):
    nblk = nblk_per_core * NCORE
    nbatch = SLAB // eb
    mesh = plsc.VectorSubcoreMesh(core_axis_name="c", subcore_axis_name="s",
                                  num_cores=NCORE, num_subcores=NSUB)
    out_type = [jax.ShapeDtypeStruct((N, cb), dt) for _ in range(4 * nblk)]
    if with_counts:
        out_type += [jax.ShapeDtypeStruct((4, N), f32)]
    scratch = [
        pltpu.VMEM_SHARED((NPAD, cb), dt),
        pltpu.VMEM((eb, cb), dt),
        pltpu.VMEM((eb, cb), dt),
        pltpu.VMEM((eb,), i32),
        pltpu.VMEM((eb,), i32),
        pltpu.VMEM((eb,), i32),
        pltpu.VMEM((eb,), i32),
        pltpu.SemaphoreType.DMA,
        pltpu.SemaphoreType.DMA,
    ]
    if with_counts:
        scratch += [
            pltpu.VMEM_SHARED((NPAD,), f32),
            pltpu.VMEM((eb,), f32),
        ]

    def body(*refs):
        tabs = refs[0:ntab]
        edges = refs[ntab]
        zc = refs[ntab + 1]
        k = ntab + 2
        if with_counts:
            oc, zcn = refs[k], refs[k + 1]
            k += 2
        outs = refs[k:k + 4 * nblk]
        k += 4 * nblk
        if with_counts:
            cnt_out = refs[k]
            k += 1
        acc, gbuf0, gbuf1, sidx0, sidx1, didx0, didx1, sem0, sem1 = \
            refs[k:k + 9]
        if with_counts:
            cnt_acc, obuf = refs[k + 9:k + 11]

        cid = lax.axis_index("c")
        sid = lax.axis_index("s")
        if with_counts:
            pltpu.sync_copy(oc, obuf)
        base = sid * STRIPE
        ebase = sid * SLAB

        for r in range(4):
            for p in range(nblk_per_core):
                for half in range(NCORE):
                    blk = half * nblk_per_core + p
                    tab = tabs[tmap[r][blk]]
                    out = outs[r * nblk + blk]
                    do_cnt = with_counts and blk == 0

                    def do_pass(tab=tab, out=out, do_cnt=do_cnt, r=r):
                        pltpu.sync_copy(zc, gbuf0)
                        for z in range(STRIPE // eb):
                            pltpu.sync_copy(
                                gbuf0, acc.at[pl.ds(base + z * eb, eb), :])
                        rem = STRIPE % eb
                        if rem:
                            pltpu.sync_copy(
                                gbuf0.at[pl.ds(0, rem), :],
                                acc.at[pl.ds(base + STRIPE - rem, rem), :])
                        if do_cnt:
                            pltpu.sync_copy(
                                zcn, cnt_acc.at[pl.ds(base, STRIPE)])
                        plsc.subcore_barrier()

                        pltpu.sync_copy(
                            edges.at[2 * r, pl.ds(ebase, eb)], sidx0)
                        pltpu.sync_copy(
                            edges.at[2 * r + 1, pl.ds(ebase, eb)], didx0)
                        pltpu.async_copy(tab.at[sidx0], gbuf0, sem0)

                        def consume(gbuf, sidx, didx, sem):
                            pltpu.make_async_copy(
                                tab.at[sidx], gbuf, sem).wait()
                            pltpu.sync_copy(gbuf, acc.at[didx], add=True)
                            if do_cnt:
                                pltpu.sync_copy(
                                    obuf, cnt_acc.at[didx], add=True)

                        def prefetch(b, gbuf, sidx, didx, sem):
                            off = ebase + b * eb
                            pltpu.sync_copy(
                                edges.at[2 * r, pl.ds(off, eb)], sidx)
                            pltpu.sync_copy(
                                edges.at[2 * r + 1, pl.ds(off, eb)], didx)
                            pltpu.async_copy(tab.at[sidx], gbuf, sem)

                        def pair(b2, carry):
                            prefetch(2 * b2 + 1, gbuf1, sidx1, didx1, sem1)
                            consume(gbuf0, sidx0, didx0, sem0)

                            @pl.when(b2 + 1 < nbatch // 2)
                            def _():
                                prefetch(2 * b2 + 2, gbuf0, sidx0, didx0,
                                         sem0)
                            consume(gbuf1, sidx1, didx1, sem1)
                            return carry

                        lax.fori_loop(0, nbatch // 2, pair, 0)
                        plsc.subcore_barrier()

                        def flush(flen):
                            def go():
                                pltpu.sync_copy(
                                    acc.at[pl.ds(base, flen), :],
                                    out.at[pl.ds(base, flen), :])
                                if do_cnt:
                                    pltpu.sync_copy(
                                        cnt_acc.at[pl.ds(base, flen)],
                                        cnt_out.at[r, pl.ds(base, flen)])
                            return go

                        pl.when(sid < NSUB - 1)(flush(STRIPE))
                        pl.when(sid == NSUB - 1)(flush(LAST_FLUSH))
                        plsc.subcore_barrier()

                    pl.when(cid == half)(do_pass)

    return pl.kernel(body, out_type=tuple(out_type), mesh=mesh,
                     scratch_types=scratch,
                     compiler_params=pltpu.CompilerParams(
                         use_tc_tiling_on_sc=False))


HB = 64
bf16 = jnp.bfloat16
_AGG_L1 = _make_agg(1, True, _L1_TMAP, 4, HB, bf16, 400)
_AGG_L2 = _make_agg(1, False, _L2_TMAP, 8, CB, bf16, 600)

_ROWS = 1000
_GRID = N // _ROWS


def _tc1_body(a0, a1, g0, g1, ca, cb, W1a, W1b, b1,
              W2x, W2y, tx0, tx1, ty0, ty1):
    ia = 1.0 / jnp.maximum(ca[...], 1.0)
    ib = 1.0 / jnp.maximum(cb[...], 1.0)
    h = jnp.broadcast_to(b1[...], (_ROWS, D_HID))
    for k, a in enumerate((a0, a1)):
        h = h + jnp.dot(a[...].astype(f32) * ia, W1a[k * HB:(k + 1) * HB, :],
                        preferred_element_type=f32)
    for k, g in enumerate((g0, g1)):
        h = h + jnp.dot(g[...].astype(f32) * ib, W1b[k * HB:(k + 1) * HB, :],
                        preferred_element_type=f32)
    h = jnp.maximum(h, 0.0)
    tx0[...] = jnp.dot(h, W2x[:, 0:CB],
                       preferred_element_type=f32).astype(bf16)
    tx1[...] = jnp.dot(h, W2x[:, CB:2 * CB],
                       preferred_element_type=f32).astype(bf16)
    ty0[...] = jnp.dot(h, W2y[:, 0:CB],
                       preferred_element_type=f32).astype(bf16)
    ty1[...] = jnp.dot(h, W2y[:, CB:2 * CB],
                       preferred_element_type=f32).astype(bf16)


def _tc1(Aa, Ab, ca, cb, W1a, W1b, b1, W2x, W2y):
    blk = lambda i: (i, 0)
    full = lambda i: (0, 0)
    spec_a = pl.BlockSpec((_ROWS, HB), blk)
    spec_c = pl.BlockSpec((_ROWS, 1), blk)
    return pl.pallas_call(
        _tc1_body,
        grid=(_GRID,),
        in_specs=[spec_a] * 4 + [spec_c] * 2 + [
            pl.BlockSpec((D_IN, D_HID), full),
            pl.BlockSpec((D_IN, D_HID), full),
            pl.BlockSpec((1, D_HID), full),
            pl.BlockSpec((D_HID, D_EMB), full),
            pl.BlockSpec((D_HID, D_EMB), full),
        ],
        out_specs=[pl.BlockSpec((_ROWS, CB), blk)] * 4,
        out_shape=[jax.ShapeDtypeStruct((N, CB), bf16) for _ in range(4)],
    )(*Aa, *Ab, ca, cb, W1a, W1b, b1, W2x, W2y)


def _tc2_body(x0, x1, g0, g1, ca, cb, b2, out):
    ia = 1.0 / jnp.maximum(ca[...], 1.0)
    ib = 1.0 / jnp.maximum(cb[...], 1.0)
    out[...] = jnp.concatenate(
        [x0[...].astype(f32) * ia + g0[...].astype(f32) * ib,
         x1[...].astype(f32) * ia + g1[...].astype(f32) * ib],
        axis=1) + b2[...]


def _tc2(Ax, Ag, ca, cb, b2):
    blk = lambda i: (i, 0)
    full = lambda i: (0, 0)
    spec_a = pl.BlockSpec((_ROWS, CB), blk)
    spec_c = pl.BlockSpec((_ROWS, 1), blk)
    return pl.pallas_call(
        _tc2_body,
        grid=(_GRID,),
        in_specs=[spec_a] * 4 + [spec_c] * 2 + [pl.BlockSpec((1, D_EMB), full)],
        out_specs=pl.BlockSpec((_ROWS, D_EMB), blk),
        out_shape=jax.ShapeDtypeStruct((N, D_EMB), f32),
    )(*Ax, *Ag, ca, cb, b2)


def kernel(x_drug, x_gene, src_dd, dst_dd, src_dg, dst_dg, src_gd, dst_gd,
           src_gg, dst_gg, W1_dd, W1_dg, W1_gd, W1_gg, b1_drug, b1_gene,
           W2_dd, W2_dg, W2_gd, W2_gg, b2_drug, b2_gene):
    pad = EP - E
    spad = jnp.zeros((pad,), i32)
    dpad = jnp.full((pad,), DUMP, i32)
    erows = []
    for s, d in ((src_dd, dst_dd), (src_dg, dst_dg),
                 (src_gd, dst_gd), (src_gg, dst_gg)):
        erows.append(jnp.concatenate([s.astype(i32), spad]))
        erows.append(jnp.concatenate([d.astype(i32), dpad]))
    edges = jnp.stack(erows)

    xd16 = x_drug.astype(bf16)
    xg16 = x_gene.astype(bf16)
    xblocks = [xd16[:, :HB], xd16[:, HB:], xg16[:, :HB], xg16[:, HB:]]

    zc1 = jnp.zeros((400, HB), bf16)
    zc2 = jnp.zeros((600, CB), bf16)
    oc = jnp.ones((400,), f32)
    zcn = jnp.zeros((STRIPE,), f32)

    outs = _AGG_L1(*xblocks, edges, zc1, oc, zcn)
    acc1 = [outs[2 * r:2 * (r + 1)] for r in range(4)]
    cnts = [outs[8][r].reshape(N, 1) for r in range(4)]

    t2_dd0, t2_dd1, t2_dg0, t2_dg1 = _tc1(
        acc1[0], acc1[2], cnts[0], cnts[2], W1_dd, W1_gd,
        b1_drug.reshape(1, D_HID), W2_dd, W2_dg)
    t2_gd0, t2_gd1, t2_gg0, t2_gg1 = _tc1(
        acc1[1], acc1[3], cnts[1], cnts[3], W1_dg, W1_gg,
        b1_gene.reshape(1, D_HID), W2_gd, W2_gg)

    outs2 = _AGG_L2(t2_dd0, t2_dd1, t2_dg0, t2_dg1,
                    t2_gd0, t2_gd1, t2_gg0, t2_gg1, edges, zc2)
    acc2 = [outs2[2 * r:2 * (r + 1)] for r in range(4)]

    o_drug = _tc2(acc2[0], acc2[2], cnts[0], cnts[2],
                  b2_drug.reshape(1, D_EMB))
    o_gene = _tc2(acc2[1], acc2[3], cnts[1], cnts[3],
                  b2_gene.reshape(1, D_EMB))
    return (o_drug, o_gene)


# speedup vs baseline: 4.2512x; 1.1348x over previous
import functools

import jax
import jax.numpy as jnp
from jax import lax
from jax.experimental import pallas as pl
from jax.experimental.pallas import tpu as pltpu
from jax.experimental.pallas import tpu_sc as plsc

N = 50000
E = 150000
D_IN = 128
D_HID = 128
D_EMB = 64
CB = 32

NSUB = 16
NCORE = 2
NPAD = 50048
STRIPE = NPAD // NSUB
LAST_FLUSH = N - (NSUB - 1) * STRIPE
DUMP = N
EB = 320
NBATCH = 30
SLAB = EB * NBATCH
EP = SLAB * NSUB

f32 = jnp.float32
i32 = jnp.int32

_L1A = ((0, (0, 1)), (2, (2, 3)))
_L1B = ((1, (0, 1)), (3, (2, 3)))
_L2A = ((0, (0, 1)), (1, (2, 3)))
_L2B = ((2, (0, 1)), (3, (2, 3)))


def _make_agg(nblk_per_core, with_counts, rel_tabs, ntab, cb, dt, eb):
    nblk = nblk_per_core * NCORE
    nrel = len(rel_tabs)
    nbatch = SLAB // eb
    mesh = plsc.VectorSubcoreMesh(core_axis_name="c", subcore_axis_name="s",
                                  num_cores=NCORE, num_subcores=NSUB)
    out_type = [jax.ShapeDtypeStruct((N, cb), dt)
                for _ in range(nrel * nblk)]
    if with_counts:
        out_type += [jax.ShapeDtypeStruct((nrel, N), f32)]
    scratch = [
        pltpu.VMEM_SHARED((NPAD, cb), dt),
        pltpu.VMEM((eb, cb), dt),
        pltpu.VMEM((eb, cb), dt),
        pltpu.VMEM((eb,), i32),
        pltpu.VMEM((eb,), i32),
        pltpu.VMEM((eb,), i32),
        pltpu.VMEM((eb,), i32),
        pltpu.SemaphoreType.DMA,
        pltpu.SemaphoreType.DMA,
    ]
    if with_counts:
        scratch += [
            pltpu.VMEM_SHARED((NPAD,), f32),
            pltpu.VMEM((eb,), f32),
        ]

    def body(*refs):
        tabs = refs[0:ntab]
        edges = refs[ntab]
        zc = refs[ntab + 1]
        k = ntab + 2
        if with_counts:
            oc, zcn = refs[k], refs[k + 1]
            k += 2
        outs = refs[k:k + nrel * nblk]
        k += nrel * nblk
        if with_counts:
            cnt_out = refs[k]
            k += 1
        acc, gbuf0, gbuf1, sidx0, sidx1, didx0, didx1, sem0, sem1 = \
            refs[k:k + 9]
        if with_counts:
            cnt_acc, obuf = refs[k + 9:k + 11]

        cid = lax.axis_index("c")
        sid = lax.axis_index("s")
        if with_counts:
            pltpu.sync_copy(oc, obuf)
        base = sid * STRIPE
        ebase = sid * SLAB

        for ri, (r, tabidx) in enumerate(rel_tabs):
            for p in range(nblk_per_core):
                for half in range(NCORE):
                    blk = half * nblk_per_core + p
                    tab = tabs[tabidx[blk]]
                    out = outs[ri * nblk + blk]
                    do_cnt = with_counts and blk == 0

                    def do_pass(tab=tab, out=out, do_cnt=do_cnt, r=r, ri=ri):
                        pltpu.sync_copy(zc, gbuf0)
                        for z in range(STRIPE // eb):
                            pltpu.sync_copy(
                                gbuf0, acc.at[pl.ds(base + z * eb, eb), :])
                        rem = STRIPE % eb
                        if rem:
                            pltpu.sync_copy(
                                gbuf0.at[pl.ds(0, rem), :],
                                acc.at[pl.ds(base + STRIPE - rem, rem), :])
                        if do_cnt:
                            pltpu.sync_copy(
                                zcn, cnt_acc.at[pl.ds(base, STRIPE)])
                        plsc.subcore_barrier()

                        pltpu.sync_copy(
                            edges.at[2 * r, pl.ds(ebase, eb)], sidx0)
                        pltpu.sync_copy(
                            edges.at[2 * r + 1, pl.ds(ebase, eb)], didx0)
                        pltpu.async_copy(tab.at[sidx0], gbuf0, sem0)

                        def consume(gbuf, sidx, didx, sem):
                            pltpu.make_async_copy(
                                tab.at[sidx], gbuf, sem).wait()
                            pltpu.sync_copy(gbuf, acc.at[didx], add=True)
                            if do_cnt:
                                pltpu.sync_copy(
                                    obuf, cnt_acc.at[didx], add=True)

                        def prefetch(b, gbuf, sidx, didx, sem):
                            off = ebase + b * eb
                            pltpu.sync_copy(
                                edges.at[2 * r, pl.ds(off, eb)], sidx)
                            pltpu.sync_copy(
                                edges.at[2 * r + 1, pl.ds(off, eb)], didx)
                            pltpu.async_copy(tab.at[sidx], gbuf, sem)

                        def pair(b2, carry):
                            prefetch(2 * b2 + 1, gbuf1, sidx1, didx1, sem1)
                            consume(gbuf0, sidx0, didx0, sem0)

                            @pl.when(b2 + 1 < nbatch // 2)
                            def _():
                                prefetch(2 * b2 + 2, gbuf0, sidx0, didx0,
                                         sem0)
                            consume(gbuf1, sidx1, didx1, sem1)
                            return carry

                        lax.fori_loop(0, nbatch // 2, pair, 0)
                        plsc.subcore_barrier()

                        def flush(flen):
                            def go():
                                pltpu.sync_copy(
                                    acc.at[pl.ds(base, flen), :],
                                    out.at[pl.ds(base, flen), :])
                                if do_cnt:
                                    pltpu.sync_copy(
                                        cnt_acc.at[pl.ds(base, flen)],
                                        cnt_out.at[ri, pl.ds(base, flen)])
                            return go

                        pl.when(sid < NSUB - 1)(flush(STRIPE))
                        pl.when(sid == NSUB - 1)(flush(LAST_FLUSH))
                        plsc.subcore_barrier()

                    pl.when(cid == half)(do_pass)

    return pl.kernel(body, out_type=tuple(out_type), mesh=mesh,
                     scratch_types=scratch,
                     compiler_params=pltpu.CompilerParams(
                         use_tc_tiling_on_sc=False))


HB = 64
bf16 = jnp.bfloat16
_AGG_L1A = _make_agg(1, True, _L1A, 4, HB, bf16, 400)
_AGG_L1B = _make_agg(1, True, _L1B, 4, HB, bf16, 400)
_AGG_L2A = _make_agg(1, False, _L2A, 4, CB, bf16, 600)
_AGG_L2B = _make_agg(1, False, _L2B, 4, CB, bf16, 600)

_ROWS = 1000
_GRID = N // _ROWS


def _tc1_body(a0, a1, g0, g1, ca, cb, W1a, W1b, b1,
              W2x, W2y, tx0, tx1, ty0, ty1):
    ia = 1.0 / jnp.maximum(ca[...], 1.0)
    ib = 1.0 / jnp.maximum(cb[...], 1.0)
    h = jnp.broadcast_to(b1[...], (_ROWS, D_HID))
    for k, a in enumerate((a0, a1)):
        h = h + jnp.dot(a[...].astype(f32) * ia, W1a[k * HB:(k + 1) * HB, :],
                        preferred_element_type=f32)
    for k, g in enumerate((g0, g1)):
        h = h + jnp.dot(g[...].astype(f32) * ib, W1b[k * HB:(k + 1) * HB, :],
                        preferred_element_type=f32)
    h = jnp.maximum(h, 0.0)
    tx0[...] = jnp.dot(h, W2x[:, 0:CB],
                       preferred_element_type=f32).astype(bf16)
    tx1[...] = jnp.dot(h, W2x[:, CB:2 * CB],
                       preferred_element_type=f32).astype(bf16)
    ty0[...] = jnp.dot(h, W2y[:, 0:CB],
                       preferred_element_type=f32).astype(bf16)
    ty1[...] = jnp.dot(h, W2y[:, CB:2 * CB],
                       preferred_element_type=f32).astype(bf16)


def _tc1(Aa, Ab, ca, cb, W1a, W1b, b1, W2x, W2y):
    blk = lambda i: (i, 0)
    full = lambda i: (0, 0)
    spec_a = pl.BlockSpec((_ROWS, HB), blk)
    spec_c = pl.BlockSpec((_ROWS, 1), blk)
    return pl.pallas_call(
        _tc1_body,
        grid=(_GRID,),
        in_specs=[spec_a] * 4 + [spec_c] * 2 + [
            pl.BlockSpec((D_IN, D_HID), full),
            pl.BlockSpec((D_IN, D_HID), full),
            pl.BlockSpec((1, D_HID), full),
            pl.BlockSpec((D_HID, D_EMB), full),
            pl.BlockSpec((D_HID, D_EMB), full),
        ],
        out_specs=[pl.BlockSpec((_ROWS, CB), blk)] * 4,
        out_shape=[jax.ShapeDtypeStruct((N, CB), bf16) for _ in range(4)],
    )(*Aa, *Ab, ca, cb, W1a, W1b, b1, W2x, W2y)


def _tc2_body(x0, x1, g0, g1, ca, cb, b2, out):
    ia = 1.0 / jnp.maximum(ca[...], 1.0)
    ib = 1.0 / jnp.maximum(cb[...], 1.0)
    out[...] = jnp.concatenate(
        [x0[...].astype(f32) * ia + g0[...].astype(f32) * ib,
         x1[...].astype(f32) * ia + g1[...].astype(f32) * ib],
        axis=1) + b2[...]


def _tc2(Ax, Ag, ca, cb, b2):
    blk = lambda i: (i, 0)
    full = lambda i: (0, 0)
    spec_a = pl.BlockSpec((_ROWS, CB), blk)
    spec_c = pl.BlockSpec((_ROWS, 1), blk)
    return pl.pallas_call(
        _tc2_body,
        grid=(_GRID,),
        in_specs=[spec_a] * 4 + [spec_c] * 2 + [pl.BlockSpec((1, D_EMB), full)],
        out_specs=pl.BlockSpec((_ROWS, D_EMB), blk),
        out_shape=jax.ShapeDtypeStruct((N, D_EMB), f32),
    )(*Ax, *Ag, ca, cb, b2)


def kernel(x_drug, x_gene, src_dd, dst_dd, src_dg, dst_dg, src_gd, dst_gd,
           src_gg, dst_gg, W1_dd, W1_dg, W1_gd, W1_gg, b1_drug, b1_gene,
           W2_dd, W2_dg, W2_gd, W2_gg, b2_drug, b2_gene):
    pad = EP - E
    spad = jnp.zeros((pad,), i32)
    dpad = jnp.full((pad,), DUMP, i32)
    erows = []
    for s, d in ((src_dd, dst_dd), (src_dg, dst_dg),
                 (src_gd, dst_gd), (src_gg, dst_gg)):
        erows.append(jnp.concatenate([s.astype(i32), spad]))
        erows.append(jnp.concatenate([d.astype(i32), dpad]))
    edges = jnp.stack(erows)

    xd16 = x_drug.astype(bf16)
    xg16 = x_gene.astype(bf16)
    xblocks = [xd16[:, :HB], xd16[:, HB:], xg16[:, :HB], xg16[:, HB:]]

    zc1 = jnp.zeros((400, HB), bf16)
    zc2 = jnp.zeros((600, CB), bf16)
    oc = jnp.ones((400,), f32)
    zcn = jnp.zeros((STRIPE,), f32)

    outs_a = _AGG_L1A(*xblocks, edges, zc1, oc, zcn)
    outs_b = _AGG_L1B(*xblocks, edges, zc1, oc, zcn)
    acc_dd, acc_gd = outs_a[0:2], outs_a[2:4]
    acc_dg, acc_gg = outs_b[0:2], outs_b[2:4]
    cnt_dd = outs_a[4][0].reshape(N, 1)
    cnt_gd = outs_a[4][1].reshape(N, 1)
    cnt_dg = outs_b[4][0].reshape(N, 1)
    cnt_gg = outs_b[4][1].reshape(N, 1)

    t2_dd0, t2_dd1, t2_dg0, t2_dg1 = _tc1(
        acc_dd, acc_gd, cnt_dd, cnt_gd, W1_dd, W1_gd,
        b1_drug.reshape(1, D_HID), W2_dd, W2_dg)
    t2_gd0, t2_gd1, t2_gg0, t2_gg1 = _tc1(
        acc_dg, acc_gg, cnt_dg, cnt_gg, W1_dg, W1_gg,
        b1_gene.reshape(1, D_HID), W2_gd, W2_gg)

    outs2a = _AGG_L2A(t2_dd0, t2_dd1, t2_dg0, t2_dg1, edges, zc2)
    outs2b = _AGG_L2B(t2_gd0, t2_gd1, t2_gg0, t2_gg1, edges, zc2)

    o_drug = _tc2(outs2a[0:2], outs2b[0:2], cnt_dd, cnt_gd,
                  b2_drug.reshape(1, D_EMB))
    o_gene = _tc2(outs2a[2:4], outs2b[2:4], cnt_dg, cnt_gg,
                  b2_gene.reshape(1, D_EMB))
    return (o_drug, o_gene)


# speedup vs baseline: 4.2612x; 1.0023x over previous
import functools

import jax
import jax.numpy as jnp
from jax import lax
from jax.experimental import pallas as pl
from jax.experimental.pallas import tpu as pltpu
from jax.experimental.pallas import tpu_sc as plsc

N = 50000
E = 150000
D_IN = 128
D_HID = 128
D_EMB = 64
CB = 32

NSUB = 16
NCORE = 2
NPAD = 50048
STRIPE = NPAD // NSUB
LAST_FLUSH = N - (NSUB - 1) * STRIPE
DUMP = N
EB = 320
NBATCH = 30
SLAB = EB * NBATCH
EP = SLAB * NSUB

f32 = jnp.float32
i32 = jnp.int32

_L1A = ((0, (0, 1)), (2, (2, 3)))
_L1B = ((1, (0, 1)), (3, (2, 3)))
_L2A = ((0, (0, 1)), (1, (2, 3)))
_L2B = ((2, (0, 1)), (3, (2, 3)))


def _make_agg(nblk_per_core, with_counts, rel_tabs, ntab, cb, dt, eb):
    nblk = nblk_per_core * NCORE
    nrel = len(rel_tabs)
    nbatch = SLAB // eb
    mesh = plsc.VectorSubcoreMesh(core_axis_name="c", subcore_axis_name="s",
                                  num_cores=NCORE, num_subcores=NSUB)
    out_type = [jax.ShapeDtypeStruct((N, cb), dt)
                for _ in range(nrel * nblk)]
    if with_counts:
        out_type += [jax.ShapeDtypeStruct((nrel, N), f32)]
    scratch = [
        pltpu.VMEM_SHARED((NPAD, cb), dt),
        pltpu.VMEM((eb, cb), dt),
        pltpu.VMEM((eb, cb), dt),
        pltpu.VMEM((eb,), i32),
        pltpu.VMEM((eb,), i32),
        pltpu.VMEM((eb,), i32),
        pltpu.VMEM((eb,), i32),
        pltpu.SemaphoreType.DMA,
        pltpu.SemaphoreType.DMA,
    ]
    if with_counts:
        scratch += [
            pltpu.VMEM_SHARED((NPAD,), f32),
            pltpu.VMEM((eb,), f32),
        ]

    def body(*refs):
        tabs = refs[0:ntab]
        edges = refs[ntab]
        zc = refs[ntab + 1]
        k = ntab + 2
        if with_counts:
            oc, zcn = refs[k], refs[k + 1]
            k += 2
        outs = refs[k:k + nrel * nblk]
        k += nrel * nblk
        if with_counts:
            cnt_out = refs[k]
            k += 1
        acc, gbuf0, gbuf1, sidx0, sidx1, didx0, didx1, sem0, sem1 = \
            refs[k:k + 9]
        if with_counts:
            cnt_acc, obuf = refs[k + 9:k + 11]

        cid = lax.axis_index("c")
        sid = lax.axis_index("s")
        if with_counts:
            pltpu.sync_copy(oc, obuf)
        base = sid * STRIPE
        ebase = sid * SLAB

        for ri, (r, tabidx) in enumerate(rel_tabs):
            for p in range(nblk_per_core):
                for half in range(NCORE):
                    blk = half * nblk_per_core + p
                    tab = tabs[tabidx[blk]]
                    out = outs[ri * nblk + blk]
                    do_cnt = with_counts and blk == 0

                    def do_pass(tab=tab, out=out, do_cnt=do_cnt, r=r, ri=ri):
                        pltpu.sync_copy(zc, gbuf0)
                        for z in range(STRIPE // eb):
                            pltpu.sync_copy(
                                gbuf0, acc.at[pl.ds(base + z * eb, eb), :])
                        rem = STRIPE % eb
                        if rem:
                            pltpu.sync_copy(
                                gbuf0.at[pl.ds(0, rem), :],
                                acc.at[pl.ds(base + STRIPE - rem, rem), :])
                        if do_cnt:
                            pltpu.sync_copy(
                                zcn, cnt_acc.at[pl.ds(base, STRIPE)])
                        plsc.subcore_barrier()

                        pltpu.sync_copy(
                            edges.at[2 * r, pl.ds(ebase, eb)], sidx0)
                        pltpu.sync_copy(
                            edges.at[2 * r + 1, pl.ds(ebase, eb)], didx0)
                        pltpu.async_copy(tab.at[sidx0], gbuf0, sem0)

                        def consume(gbuf, sidx, didx, sem):
                            pltpu.make_async_copy(
                                tab.at[sidx], gbuf, sem).wait()
                            pltpu.sync_copy(gbuf, acc.at[didx], add=True)
                            if do_cnt:
                                pltpu.sync_copy(
                                    obuf, cnt_acc.at[didx], add=True)

                        def prefetch(b, gbuf, sidx, didx, sem):
                            off = ebase + b * eb
                            pltpu.sync_copy(
                                edges.at[2 * r, pl.ds(off, eb)], sidx)
                            pltpu.sync_copy(
                                edges.at[2 * r + 1, pl.ds(off, eb)], didx)
                            pltpu.async_copy(tab.at[sidx], gbuf, sem)

                        def pair(b2, carry):
                            prefetch(2 * b2 + 1, gbuf1, sidx1, didx1, sem1)
                            consume(gbuf0, sidx0, didx0, sem0)

                            @pl.when(b2 + 1 < nbatch // 2)
                            def _():
                                prefetch(2 * b2 + 2, gbuf0, sidx0, didx0,
                                         sem0)
                            consume(gbuf1, sidx1, didx1, sem1)
                            return carry

                        lax.fori_loop(0, nbatch // 2, pair, 0)
                        plsc.subcore_barrier()

                        def flush(flen):
                            def go():
                                pltpu.sync_copy(
                                    acc.at[pl.ds(base, flen), :],
                                    out.at[pl.ds(base, flen), :])
                                if do_cnt:
                                    pltpu.sync_copy(
                                        cnt_acc.at[pl.ds(base, flen)],
                                        cnt_out.at[ri, pl.ds(base, flen)])
                            return go

                        pl.when(sid < NSUB - 1)(flush(STRIPE))
                        pl.when(sid == NSUB - 1)(flush(LAST_FLUSH))
                        plsc.subcore_barrier()

                    pl.when(cid == half)(do_pass)

    return pl.kernel(body, out_type=tuple(out_type), mesh=mesh,
                     scratch_types=scratch,
                     compiler_params=pltpu.CompilerParams(
                         use_tc_tiling_on_sc=False))


HB = 64
bf16 = jnp.bfloat16
_AGG_L1A = _make_agg(1, True, _L1A, 4, HB, bf16, 400)
_AGG_L1B = _make_agg(1, True, _L1B, 4, HB, bf16, 400)
_AGG_L2A = _make_agg(1, False, _L2A, 4, CB, bf16, 1200)
_AGG_L2B = _make_agg(1, False, _L2B, 4, CB, bf16, 1200)

_ROWS = 1000
_GRID = N // _ROWS


def _tc1_body(a0, a1, g0, g1, ca, cb, W1a, W1b, b1,
              W2x, W2y, tx0, tx1, ty0, ty1):
    ia = 1.0 / jnp.maximum(ca[...], 1.0)
    ib = 1.0 / jnp.maximum(cb[...], 1.0)
    h = jnp.broadcast_to(b1[...], (_ROWS, D_HID))
    for k, a in enumerate((a0, a1)):
        h = h + jnp.dot(a[...].astype(f32) * ia, W1a[k * HB:(k + 1) * HB, :],
                        preferred_element_type=f32)
    for k, g in enumerate((g0, g1)):
        h = h + jnp.dot(g[...].astype(f32) * ib, W1b[k * HB:(k + 1) * HB, :],
                        preferred_element_type=f32)
    h = jnp.maximum(h, 0.0)
    tx0[...] = jnp.dot(h, W2x[:, 0:CB],
                       preferred_element_type=f32).astype(bf16)
    tx1[...] = jnp.dot(h, W2x[:, CB:2 * CB],
                       preferred_element_type=f32).astype(bf16)
    ty0[...] = jnp.dot(h, W2y[:, 0:CB],
                       preferred_element_type=f32).astype(bf16)
    ty1[...] = jnp.dot(h, W2y[:, CB:2 * CB],
                       preferred_element_type=f32).astype(bf16)


def _tc1(Aa, Ab, ca, cb, W1a, W1b, b1, W2x, W2y):
    blk = lambda i: (i, 0)
    full = lambda i: (0, 0)
    spec_a = pl.BlockSpec((_ROWS, HB), blk)
    spec_c = pl.BlockSpec((_ROWS, 1), blk)
    return pl.pallas_call(
        _tc1_body,
        grid=(_GRID,),
        in_specs=[spec_a] * 4 + [spec_c] * 2 + [
            pl.BlockSpec((D_IN, D_HID), full),
            pl.BlockSpec((D_IN, D_HID), full),
            pl.BlockSpec((1, D_HID), full),
            pl.BlockSpec((D_HID, D_EMB), full),
            pl.BlockSpec((D_HID, D_EMB), full),
        ],
        out_specs=[pl.BlockSpec((_ROWS, CB), blk)] * 4,
        out_shape=[jax.ShapeDtypeStruct((N, CB), bf16) for _ in range(4)],
    )(*Aa, *Ab, ca, cb, W1a, W1b, b1, W2x, W2y)


def _tc2_body(x0, x1, g0, g1, ca, cb, b2, out):
    ia = 1.0 / jnp.maximum(ca[...], 1.0)
    ib = 1.0 / jnp.maximum(cb[...], 1.0)
    out[...] = jnp.concatenate(
        [x0[...].astype(f32) * ia + g0[...].astype(f32) * ib,
         x1[...].astype(f32) * ia + g1[...].astype(f32) * ib],
        axis=1) + b2[...]


def _tc2(Ax, Ag, ca, cb, b2):
    blk = lambda i: (i, 0)
    full = lambda i: (0, 0)
    spec_a = pl.BlockSpec((_ROWS, CB), blk)
    spec_c = pl.BlockSpec((_ROWS, 1), blk)
    return pl.pallas_call(
        _tc2_body,
        grid=(_GRID,),
        in_specs=[spec_a] * 4 + [spec_c] * 2 + [pl.BlockSpec((1, D_EMB), full)],
        out_specs=pl.BlockSpec((_ROWS, D_EMB), blk),
        out_shape=jax.ShapeDtypeStruct((N, D_EMB), f32),
    )(*Ax, *Ag, ca, cb, b2)


def kernel(x_drug, x_gene, src_dd, dst_dd, src_dg, dst_dg, src_gd, dst_gd,
           src_gg, dst_gg, W1_dd, W1_dg, W1_gd, W1_gg, b1_drug, b1_gene,
           W2_dd, W2_dg, W2_gd, W2_gg, b2_drug, b2_gene):
    pad = EP - E
    spad = jnp.zeros((pad,), i32)
    dpad = jnp.full((pad,), DUMP, i32)
    erows = []
    for s, d in ((src_dd, dst_dd), (src_dg, dst_dg),
                 (src_gd, dst_gd), (src_gg, dst_gg)):
        erows.append(jnp.concatenate([s.astype(i32), spad]))
        erows.append(jnp.concatenate([d.astype(i32), dpad]))
    edges = jnp.stack(erows)

    xd16 = x_drug.astype(bf16)
    xg16 = x_gene.astype(bf16)
    xblocks = [xd16[:, :HB], xd16[:, HB:], xg16[:, :HB], xg16[:, HB:]]

    zc1 = jnp.zeros((400, HB), bf16)
    zc2 = jnp.zeros((1200, CB), bf16)
    oc = jnp.ones((400,), f32)
    zcn = jnp.zeros((STRIPE,), f32)

    outs_a = _AGG_L1A(*xblocks, edges, zc1, oc, zcn)
    outs_b = _AGG_L1B(*xblocks, edges, zc1, oc, zcn)
    acc_dd, acc_gd = outs_a[0:2], outs_a[2:4]
    acc_dg, acc_gg = outs_b[0:2], outs_b[2:4]
    cnt_dd = outs_a[4][0].reshape(N, 1)
    cnt_gd = outs_a[4][1].reshape(N, 1)
    cnt_dg = outs_b[4][0].reshape(N, 1)
    cnt_gg = outs_b[4][1].reshape(N, 1)

    t2_dd0, t2_dd1, t2_dg0, t2_dg1 = _tc1(
        acc_dd, acc_gd, cnt_dd, cnt_gd, W1_dd, W1_gd,
        b1_drug.reshape(1, D_HID), W2_dd, W2_dg)
    outs2a = _AGG_L2A(t2_dd0, t2_dd1, t2_dg0, t2_dg1, edges, zc2)
    t2_gd0, t2_gd1, t2_gg0, t2_gg1 = _tc1(
        acc_dg, acc_gg, cnt_dg, cnt_gg, W1_dg, W1_gg,
        b1_gene.reshape(1, D_HID), W2_gd, W2_gg)
    outs2b = _AGG_L2B(t2_gd0, t2_gd1, t2_gg0, t2_gg1, edges, zc2)

    o_drug = _tc2(outs2a[0:2], outs2b[0:2], cnt_dd, cnt_gd,
                  b2_drug.reshape(1, D_EMB))
    o_gene = _tc2(outs2a[2:4], outs2b[2:4], cnt_dg, cnt_gg,
                  b2_gene.reshape(1, D_EMB))
    return (o_drug, o_gene)


# speedup vs baseline: 4.3272x; 1.0155x over previous
import functools

import jax
import jax.numpy as jnp
from jax import lax
from jax.experimental import pallas as pl
from jax.experimental.pallas import tpu as pltpu
from jax.experimental.pallas import tpu_sc as plsc

N = 50000
E = 150000
D_IN = 128
D_HID = 128
D_EMB = 64
CB = 32

NSUB = 16
NCORE = 2
NPAD = 50048
STRIPE = NPAD // NSUB
LAST_FLUSH = N - (NSUB - 1) * STRIPE
DUMP = N
EB = 320
NBATCH = 30
SLAB = EB * NBATCH
EP = SLAB * NSUB

f32 = jnp.float32
i32 = jnp.int32

_L1A = ((0, 1),)
_L1B = ((0, 1),)
_L1C = ((0, 1), (2, 3))
_L2A = ((0, 1), (2, 3))
_L2B = ((0, 1), (2, 3))


def _make_agg(nblk_per_core, with_counts, rel_tabs, ntab, cb, dt, eb):
    nblk = nblk_per_core * NCORE
    nrel = len(rel_tabs)
    nbatch = SLAB // eb
    mesh = plsc.VectorSubcoreMesh(core_axis_name="c", subcore_axis_name="s",
                                  num_cores=NCORE, num_subcores=NSUB)
    out_type = [jax.ShapeDtypeStruct((N, cb), dt)
                for _ in range(nrel * nblk)]
    if with_counts:
        out_type += [jax.ShapeDtypeStruct((nrel, N), f32)]
    scratch = [
        pltpu.VMEM_SHARED((NPAD, cb), dt),
        pltpu.VMEM((eb, cb), dt),
        pltpu.VMEM((eb, cb), dt),
        pltpu.VMEM((eb,), i32),
        pltpu.VMEM((eb,), i32),
        pltpu.VMEM((eb,), i32),
        pltpu.VMEM((eb,), i32),
        pltpu.SemaphoreType.DMA,
        pltpu.SemaphoreType.DMA,
    ]
    if with_counts:
        scratch += [
            pltpu.VMEM_SHARED((NPAD,), f32),
            pltpu.VMEM((eb,), f32),
        ]

    def body(*refs):
        tabs = refs[0:ntab]
        eds = refs[ntab:ntab + nrel]
        zc = refs[ntab + nrel]
        k = ntab + nrel + 1
        if with_counts:
            oc, zcn = refs[k], refs[k + 1]
            k += 2
        outs = refs[k:k + nrel * nblk]
        k += nrel * nblk
        if with_counts:
            cnt_out = refs[k]
            k += 1
        acc, gbuf0, gbuf1, sidx0, sidx1, didx0, didx1, sem0, sem1 = \
            refs[k:k + 9]
        if with_counts:
            cnt_acc, obuf = refs[k + 9:k + 11]

        cid = lax.axis_index("c")
        sid = lax.axis_index("s")
        if with_counts:
            pltpu.sync_copy(oc, obuf)
        base = sid * STRIPE
        ebase = sid * SLAB

        for ri, tabidx in enumerate(rel_tabs):
            for p in range(nblk_per_core):
                for half in range(NCORE):
                    blk = half * nblk_per_core + p
                    tab = tabs[tabidx[blk]]
                    out = outs[ri * nblk + blk]
                    do_cnt = with_counts and blk == 0

                    def do_pass(tab=tab, out=out, do_cnt=do_cnt,
                                ed=eds[ri], ri=ri):
                        pltpu.sync_copy(zc, gbuf0)
                        for z in range(STRIPE // eb):
                            pltpu.sync_copy(
                                gbuf0, acc.at[pl.ds(base + z * eb, eb), :])
                        rem = STRIPE % eb
                        if rem:
                            pltpu.sync_copy(
                                gbuf0.at[pl.ds(0, rem), :],
                                acc.at[pl.ds(base + STRIPE - rem, rem), :])
                        if do_cnt:
                            pltpu.sync_copy(
                                zcn, cnt_acc.at[pl.ds(base, STRIPE)])
                        plsc.subcore_barrier()

                        pltpu.sync_copy(ed.at[0, pl.ds(ebase, eb)], sidx0)
                        pltpu.sync_copy(ed.at[1, pl.ds(ebase, eb)], didx0)
                        pltpu.async_copy(tab.at[sidx0], gbuf0, sem0)

                        def consume(gbuf, sidx, didx, sem):
                            pltpu.make_async_copy(
                                tab.at[sidx], gbuf, sem).wait()
                            pltpu.sync_copy(gbuf, acc.at[didx], add=True)
                            if do_cnt:
                                pltpu.sync_copy(
                                    obuf, cnt_acc.at[didx], add=True)

                        def prefetch(b, gbuf, sidx, didx, sem):
                            off = ebase + b * eb
                            pltpu.sync_copy(
                                ed.at[0, pl.ds(off, eb)], sidx)
                            pltpu.sync_copy(
                                ed.at[1, pl.ds(off, eb)], didx)
                            pltpu.async_copy(tab.at[sidx], gbuf, sem)

                        def pair(b2, carry):
                            prefetch(2 * b2 + 1, gbuf1, sidx1, didx1, sem1)
                            consume(gbuf0, sidx0, didx0, sem0)

                            @pl.when(b2 + 1 < nbatch // 2)
                            def _():
                                prefetch(2 * b2 + 2, gbuf0, sidx0, didx0,
                                         sem0)
                            consume(gbuf1, sidx1, didx1, sem1)
                            return carry

                        lax.fori_loop(0, nbatch // 2, pair, 0)
                        plsc.subcore_barrier()

                        def flush(flen):
                            def go():
                                pltpu.sync_copy(
                                    acc.at[pl.ds(base, flen), :],
                                    out.at[pl.ds(base, flen), :])
                                if do_cnt:
                                    pltpu.sync_copy(
                                        cnt_acc.at[pl.ds(base, flen)],
                                        cnt_out.at[ri, pl.ds(base, flen)])
                            return go

                        pl.when(sid < NSUB - 1)(flush(STRIPE))
                        pl.when(sid == NSUB - 1)(flush(LAST_FLUSH))
                        plsc.subcore_barrier()

                    pl.when(cid == half)(do_pass)

    return pl.kernel(body, out_type=tuple(out_type), mesh=mesh,
                     scratch_types=scratch,
                     compiler_params=pltpu.CompilerParams(
                         use_tc_tiling_on_sc=False))


HB = 64
bf16 = jnp.bfloat16
_AGG_L1A = _make_agg(1, True, _L1A, 2, HB, bf16, 400)
_AGG_L1B = _make_agg(1, True, _L1B, 2, HB, bf16, 400)
_AGG_L1C = _make_agg(1, True, _L1C, 4, HB, bf16, 400)
_AGG_L2A = _make_agg(1, False, _L2A, 4, CB, bf16, 1200)
_AGG_L2B = _make_agg(1, False, _L2B, 4, CB, bf16, 1200)

_ROWS = 1000
_GRID = N // _ROWS


def _tc1_body(a0, a1, g0, g1, ca, cb, W1a, W1b, b1,
              W2x, W2y, tx0, tx1, ty0, ty1):
    ia = 1.0 / jnp.maximum(ca[...], 1.0)
    ib = 1.0 / jnp.maximum(cb[...], 1.0)
    h = jnp.broadcast_to(b1[...], (_ROWS, D_HID))
    for k, a in enumerate((a0, a1)):
        h = h + jnp.dot(a[...].astype(f32) * ia, W1a[k * HB:(k + 1) * HB, :],
                        preferred_element_type=f32)
    for k, g in enumerate((g0, g1)):
        h = h + jnp.dot(g[...].astype(f32) * ib, W1b[k * HB:(k + 1) * HB, :],
                        preferred_element_type=f32)
    h = jnp.maximum(h, 0.0)
    tx0[...] = jnp.dot(h, W2x[:, 0:CB],
                       preferred_element_type=f32).astype(bf16)
    tx1[...] = jnp.dot(h, W2x[:, CB:2 * CB],
                       preferred_element_type=f32).astype(bf16)
    ty0[...] = jnp.dot(h, W2y[:, 0:CB],
                       preferred_element_type=f32).astype(bf16)
    ty1[...] = jnp.dot(h, W2y[:, CB:2 * CB],
                       preferred_element_type=f32).astype(bf16)


def _tc1(Aa, Ab, ca, cb, W1a, W1b, b1, W2x, W2y):
    blk = lambda i: (i, 0)
    full = lambda i: (0, 0)
    spec_a = pl.BlockSpec((_ROWS, HB), blk)
    spec_c = pl.BlockSpec((_ROWS, 1), blk)
    return pl.pallas_call(
        _tc1_body,
        grid=(_GRID,),
        in_specs=[spec_a] * 4 + [spec_c] * 2 + [
            pl.BlockSpec((D_IN, D_HID), full),
            pl.BlockSpec((D_IN, D_HID), full),
            pl.BlockSpec((1, D_HID), full),
            pl.BlockSpec((D_HID, D_EMB), full),
            pl.BlockSpec((D_HID, D_EMB), full),
        ],
        out_specs=[pl.BlockSpec((_ROWS, CB), blk)] * 4,
        out_shape=[jax.ShapeDtypeStruct((N, CB), bf16) for _ in range(4)],
    )(*Aa, *Ab, ca, cb, W1a, W1b, b1, W2x, W2y)


def _tc2_body(x0, x1, g0, g1, ca, cb, b2, out):
    ia = 1.0 / jnp.maximum(ca[...], 1.0)
    ib = 1.0 / jnp.maximum(cb[...], 1.0)
    out[...] = jnp.concatenate(
        [x0[...].astype(f32) * ia + g0[...].astype(f32) * ib,
         x1[...].astype(f32) * ia + g1[...].astype(f32) * ib],
        axis=1) + b2[...]


def _tc2(Ax, Ag, ca, cb, b2):
    blk = lambda i: (i, 0)
    full = lambda i: (0, 0)
    spec_a = pl.BlockSpec((_ROWS, CB), blk)
    spec_c = pl.BlockSpec((_ROWS, 1), blk)
    return pl.pallas_call(
        _tc2_body,
        grid=(_GRID,),
        in_specs=[spec_a] * 4 + [spec_c] * 2 + [pl.BlockSpec((1, D_EMB), full)],
        out_specs=pl.BlockSpec((_ROWS, D_EMB), blk),
        out_shape=jax.ShapeDtypeStruct((N, D_EMB), f32),
    )(*Ax, *Ag, ca, cb, b2)


def kernel(x_drug, x_gene, src_dd, dst_dd, src_dg, dst_dg, src_gd, dst_gd,
           src_gg, dst_gg, W1_dd, W1_dg, W1_gd, W1_gg, b1_drug, b1_gene,
           W2_dd, W2_dg, W2_gd, W2_gg, b2_drug, b2_gene):
    pad = EP - E
    spad = jnp.zeros((pad,), i32)
    dpad = jnp.full((pad,), DUMP, i32)
    e_dd, e_dg, e_gd, e_gg = (
        jnp.stack([jnp.concatenate([s.astype(i32), spad]),
                   jnp.concatenate([d.astype(i32), dpad])])
        for s, d in ((src_dd, dst_dd), (src_dg, dst_dg),
                     (src_gd, dst_gd), (src_gg, dst_gg)))

    xd16 = x_drug.astype(bf16)
    xg16 = x_gene.astype(bf16)
    xblocks = [xd16[:, :HB], xd16[:, HB:], xg16[:, :HB], xg16[:, HB:]]

    zc1 = jnp.zeros((400, HB), bf16)
    zc2 = jnp.zeros((1200, CB), bf16)
    oc = jnp.ones((400,), f32)
    zcn = jnp.zeros((STRIPE,), f32)

    outs_a = _AGG_L1A(xblocks[0], xblocks[1], e_dd, zc1, oc, zcn)
    outs_b = _AGG_L1B(xblocks[2], xblocks[3], e_gd, zc1, oc, zcn)
    outs_c = _AGG_L1C(*xblocks, e_dg, e_gg, zc1, oc, zcn)
    acc_dd, cnt_dd = outs_a[0:2], outs_a[2][0].reshape(N, 1)
    acc_gd, cnt_gd = outs_b[0:2], outs_b[2][0].reshape(N, 1)
    acc_dg, acc_gg = outs_c[0:2], outs_c[2:4]
    cnt_dg = outs_c[4][0].reshape(N, 1)
    cnt_gg = outs_c[4][1].reshape(N, 1)

    t2_dd0, t2_dd1, t2_dg0, t2_dg1 = _tc1(
        acc_dd, acc_gd, cnt_dd, cnt_gd, W1_dd, W1_gd,
        b1_drug.reshape(1, D_HID), W2_dd, W2_dg)
    outs2a = _AGG_L2A(t2_dd0, t2_dd1, t2_dg0, t2_dg1, e_dd, e_dg, zc2)
    t2_gd0, t2_gd1, t2_gg0, t2_gg1 = _tc1(
        acc_dg, acc_gg, cnt_dg, cnt_gg, W1_dg, W1_gg,
        b1_gene.reshape(1, D_HID), W2_gd, W2_gg)
    outs2b = _AGG_L2B(t2_gd0, t2_gd1, t2_gg0, t2_gg1, e_gd, e_gg, zc2)

    o_drug = _tc2(outs2a[0:2], outs2b[0:2], cnt_dd, cnt_gd,
                  b2_drug.reshape(1, D_EMB))
    o_gene = _tc2(outs2a[2:4], outs2b[2:4], cnt_dg, cnt_gg,
                  b2_gene.reshape(1, D_EMB))
    return (o_drug, o_gene)
